# Initial kernel scaffold; baseline (speedup 1.0000x reference)
import jax
import jax.numpy as jnp
from jax.experimental import pallas as pl


def kernel(node_feat, angle_feat, aux_feat, edge_index, W0, b0, W1, b1, G0, g0, G1, g1, Wout, We0, be0, We1, be1, Ge0, ge0, Ge1, ge1):
    raise NotImplementedError("write your pallas kernel here")


# speedup vs baseline: 1.3553x; 1.3553x over previous
import functools

import jax
import jax.numpy as jnp
from jax import lax
from jax.experimental import pallas as pl
from jax.experimental.pallas import tpu as pltpu
from jax.experimental.pallas import tpu_sc as plsc

NC = 2
NS = 16
NW = NC * NS
CH = 128


def _sc_gather2(table, idx_i3, idx_j3):
    n_rows, d = table.shape
    nw, cpw, ch = idx_i3.shape
    e_pad = nw * cpw * ch
    mesh = plsc.VectorSubcoreMesh(core_axis_name="c", subcore_axis_name="s")

    @functools.partial(
        pl.kernel,
        out_type=(
            jax.ShapeDtypeStruct((e_pad, d), jnp.float32),
            jax.ShapeDtypeStruct((e_pad, d), jnp.float32),
        ),
        mesh=mesh,
        scratch_types=[
            pltpu.VMEM((cpw, ch), jnp.int32),
            pltpu.VMEM((cpw, ch), jnp.int32),
            pltpu.VMEM((ch, d), jnp.float32),
            pltpu.VMEM((ch, d), jnp.float32),
            pltpu.SemaphoreType.DMA,
            pltpu.SemaphoreType.DMA,
        ],
    )
    def k(table_h, idxi_h, idxj_h, oi_h, oj_h, idxi_v, idxj_v, bi_v, bj_v,
          sem_i, sem_j):
        wid = lax.axis_index("s") * NC + lax.axis_index("c")
        base = wid * cpw * ch
        pltpu.sync_copy(idxi_h.at[wid], idxi_v)
        pltpu.sync_copy(idxj_h.at[wid], idxj_v)

        def body(c, carry):
            pltpu.async_copy(table_h.at[idxi_v.at[c]], bi_v, sem_i)
            pltpu.async_copy(table_h.at[idxj_v.at[c]], bj_v, sem_j)
            pltpu.make_async_copy(table_h.at[idxi_v.at[c]], bi_v, sem_i).wait()
            pltpu.sync_copy(bi_v, oi_h.at[pl.ds(base + c * ch, ch)])
            pltpu.make_async_copy(table_h.at[idxj_v.at[c]], bj_v, sem_j).wait()
            pltpu.sync_copy(bj_v, oj_h.at[pl.ds(base + c * ch, ch)])
            return carry

        lax.fori_loop(0, cpw, body, 0)

    return k(table, idx_i3, idx_j3)


def _sc_scatter_add(msg, dst3, zeros, n_rows):
    e_pad, d = msg.shape
    nw, cpw, ch = dst3.shape
    npt = n_rows // NS
    mesh = plsc.VectorSubcoreMesh(core_axis_name="c", subcore_axis_name="s")

    @functools.partial(
        pl.kernel,
        out_type=jax.ShapeDtypeStruct((NC, n_rows, d), jnp.float32),
        mesh=mesh,
        scratch_types=[
            pltpu.VMEM((cpw, ch), jnp.int32),
            pltpu.VMEM((ch, d), jnp.float32),
            pltpu.VMEM_SHARED((n_rows, d), jnp.float32),
        ],
    )
    def k(msg_h, dst_h, zeros_h, out_h, idx_v, rows_v, acc_s):
        ci = lax.axis_index("c")
        si = lax.axis_index("s")
        wid = si * NC + ci
        pltpu.sync_copy(zeros_h.at[pl.ds(si * npt, npt)],
                        acc_s.at[pl.ds(si * npt, npt)])
        plsc.subcore_barrier()
        pltpu.sync_copy(dst_h.at[wid], idx_v)
        base = wid * cpw * ch

        def body(c, carry):
            pltpu.sync_copy(msg_h.at[pl.ds(base + c * ch, ch)], rows_v)
            pltpu.sync_copy(rows_v, acc_s.at[idx_v.at[c]], add=True)
            return carry

        lax.fori_loop(0, cpw, body, 0)
        plsc.subcore_barrier()
        pltpu.sync_copy(acc_s.at[pl.ds(si * npt, npt)],
                        out_h.at[ci, pl.ds(si * npt, npt)])

    return k(msg, dst3, zeros)


def _tc_node_mlp(bi, ang, aux, bj, wbi, wang, waux, wbj, w1t, g1t, woutt,
                 bhg, b1r, g1r, e_valid, block_rows):
    e_pad, dn = bi.shape
    da = ang.shape[1]
    dx = aux.shape[1]
    h = w1t.shape[0]
    grid = e_pad // block_rows

    def body(bi_r, ang_r, aux_r, bj_r, wbi_r, wang_r, waux_r, wbj_r, w1_r,
             g1_r, wout_r, bhg_r, b1_r, g1b_r, o_r):
        x = jnp.dot(bi_r[...], wbi_r[...], preferred_element_type=jnp.float32)
        x += jnp.dot(ang_r[...], wang_r[...], preferred_element_type=jnp.float32)
        x += jnp.dot(aux_r[...], waux_r[...], preferred_element_type=jnp.float32)
        x += jnp.dot(bj_r[...], wbj_r[...], preferred_element_type=jnp.float32)
        x += bhg_r[...]
        a = x * jax.nn.sigmoid(x)
        ah = a[:, :h]
        ag = a[:, h:]
        h2 = jnp.dot(ah, w1_r[...], preferred_element_type=jnp.float32) + b1_r[...]
        h2 = h2 * jax.nn.sigmoid(h2)
        g2 = jax.nn.sigmoid(
            jnp.dot(ag, g1_r[...], preferred_element_type=jnp.float32) + g1b_r[...])
        m = h2 * g2
        o = jnp.dot(m, wout_r[...], preferred_element_type=jnp.float32)
        rows = (pl.program_id(0) * block_rows
                + lax.broadcasted_iota(jnp.int32, (block_rows, 1), 0))
        o_r[...] = jnp.where(rows < e_valid, o, 0.0)

    full = lambda i: (0, 0)
    return pl.pallas_call(
        body,
        grid=(grid,),
        in_specs=[
            pl.BlockSpec((block_rows, dn), lambda i: (i, 0)),
            pl.BlockSpec((block_rows, da), lambda i: (i, 0)),
            pl.BlockSpec((block_rows, dx), lambda i: (i, 0)),
            pl.BlockSpec((block_rows, dn), lambda i: (i, 0)),
            pl.BlockSpec(wbi.shape, full),
            pl.BlockSpec(wang.shape, full),
            pl.BlockSpec(waux.shape, full),
            pl.BlockSpec(wbj.shape, full),
            pl.BlockSpec(w1t.shape, full),
            pl.BlockSpec(g1t.shape, full),
            pl.BlockSpec(woutt.shape, full),
            pl.BlockSpec(bhg.shape, full),
            pl.BlockSpec(b1r.shape, full),
            pl.BlockSpec(g1r.shape, full),
        ],
        out_specs=pl.BlockSpec((block_rows, dn), lambda i: (i, 0)),
        out_shape=jax.ShapeDtypeStruct((e_pad, dn), jnp.float32),
    )(bi, ang, aux, bj, wbi, wang, waux, wbj, w1t, g1t, woutt, bhg, b1r, g1r)


def _tc_edge_mlp(bi, ang, aux, bj, wbi, wang, waux, wbj, w1t, g1t, bhg, b1r,
                 g1r, block_rows):
    e_pad, dn = bi.shape
    da = ang.shape[1]
    dx = aux.shape[1]
    h = w1t.shape[0]
    grid = e_pad // block_rows

    def body(bi_r, ang_r, aux_r, bj_r, wbi_r, wang_r, waux_r, wbj_r, w1_r,
             g1_r, bhg_r, b1_r, g1b_r, o_r):
        x = jnp.dot(bi_r[...], wbi_r[...], preferred_element_type=jnp.float32)
        x += jnp.dot(ang_r[...], wang_r[...], preferred_element_type=jnp.float32)
        x += jnp.dot(aux_r[...], waux_r[...], preferred_element_type=jnp.float32)
        x += jnp.dot(bj_r[...], wbj_r[...], preferred_element_type=jnp.float32)
        x += bhg_r[...]
        a = x * jax.nn.sigmoid(x)
        ah = a[:, :h]
        ag = a[:, h:]
        h2 = jnp.dot(ah, w1_r[...], preferred_element_type=jnp.float32) + b1_r[...]
        h2 = h2 * jax.nn.sigmoid(h2)
        g2 = jax.nn.sigmoid(
            jnp.dot(ag, g1_r[...], preferred_element_type=jnp.float32) + g1b_r[...])
        o_r[...] = ang_r[...] + h2 * g2

    full = lambda i: (0, 0)
    return pl.pallas_call(
        body,
        grid=(grid,),
        in_specs=[
            pl.BlockSpec((block_rows, dn), lambda i: (i, 0)),
            pl.BlockSpec((block_rows, da), lambda i: (i, 0)),
            pl.BlockSpec((block_rows, dx), lambda i: (i, 0)),
            pl.BlockSpec((block_rows, dn), lambda i: (i, 0)),
            pl.BlockSpec(wbi.shape, full),
            pl.BlockSpec(wang.shape, full),
            pl.BlockSpec(waux.shape, full),
            pl.BlockSpec(wbj.shape, full),
            pl.BlockSpec(w1t.shape, full),
            pl.BlockSpec(g1t.shape, full),
            pl.BlockSpec(bhg.shape, full),
            pl.BlockSpec(b1r.shape, full),
            pl.BlockSpec(g1r.shape, full),
        ],
        out_specs=pl.BlockSpec((block_rows, da), lambda i: (i, 0)),
        out_shape=jax.ShapeDtypeStruct((e_pad, da), jnp.float32),
    )(bi, ang, aux, bj, wbi, wang, waux, wbj, w1t, g1t, bhg, b1r, g1r)


def _tc_add_partials(node_feat, parts, block_rows):
    n, d = node_feat.shape
    grid = n // block_rows

    def body(nf_r, p_r, o_r):
        o_r[...] = nf_r[...] + p_r[0] + p_r[1]

    return pl.pallas_call(
        body,
        grid=(grid,),
        in_specs=[
            pl.BlockSpec((block_rows, d), lambda i: (i, 0)),
            pl.BlockSpec((NC, block_rows, d), lambda i: (0, i, 0)),
        ],
        out_specs=pl.BlockSpec((block_rows, d), lambda i: (i, 0)),
        out_shape=jax.ShapeDtypeStruct((n, d), jnp.float32),
    )(node_feat, parts)


def kernel(node_feat, angle_feat, aux_feat, edge_index, W0, b0, W1, b1, G0,
           g0, G1, g1, Wout, We0, be0, We1, be1, Ge0, ge0, Ge1, ge1):
    n, dn = node_feat.shape
    e, da = angle_feat.shape
    dx = aux_feat.shape[1]
    h = W0.shape[0]

    grain = NW * CH
    cpw = -(-e // grain)
    e_pad = grain * cpw
    pad = e_pad - e

    src = edge_index[0]
    dst = edge_index[1]
    src3 = jnp.pad(src, (0, pad)).reshape(NW, cpw, CH)
    dst3 = jnp.pad(dst, (0, pad)).reshape(NW, cpw, CH)
    angp = jnp.pad(angle_feat, ((0, pad), (0, 0)))
    auxp = jnp.pad(aux_feat, ((0, pad), (0, 0)))
    n_grain = 8 * NS
    n_pad = n_grain * (-(-n // n_grain))
    zeros_n = jnp.zeros((n_pad, dn), jnp.float32)

    wg0 = jnp.concatenate([W0.T, G0.T], axis=1)
    wbi0 = wg0[:dn]
    wang0 = wg0[dn:dn + da]
    waux0 = wg0[dn + da:dn + da + dx]
    wbj0 = wg0[dn + da + dx:]
    bhg0 = jnp.concatenate([b0, g0]).reshape(1, 2 * h)
    w1t = W1.T
    g1t = G1.T
    woutt = Wout.T
    b1r = b1.reshape(1, -1)
    g1r = g1.reshape(1, -1)

    wge = jnp.concatenate([We0.T, Ge0.T], axis=1)
    wbie = wge[:dn]
    wange = wge[dn:dn + da]
    wauxe = wge[dn + da:dn + da + dx]
    wbje = wge[dn + da + dx:]
    bhge = jnp.concatenate([be0, ge0]).reshape(1, 2 * h)
    we1t = We1.T
    ge1t = Ge1.T
    be1r = be1.reshape(1, -1)
    ge1r = ge1.reshape(1, -1)

    bi, bj = _sc_gather2(node_feat, src3, dst3)
    msgw = _tc_node_mlp(bi, angp, auxp, bj, wbi0, wang0, waux0, wbj0, w1t,
                        g1t, woutt, bhg0, b1r, g1r, e, 1024)
    parts = _sc_scatter_add(msgw, dst3, zeros_n, n_pad)
    new_node = _tc_add_partials(node_feat, parts, 1000)
    bi2, bj2 = _sc_gather2(new_node, src3, dst3)
    new_edge_p = _tc_edge_mlp(bi2, angp, auxp, bj2, wbie, wange, wauxe, wbje,
                              we1t, ge1t, bhge, be1r, ge1r, 1024)
    return new_node, new_edge_p[:e]


# speedup vs baseline: 2.7563x; 2.0336x over previous
import functools

import jax
import jax.numpy as jnp
from jax import lax
from jax.experimental import pallas as pl
from jax.experimental.pallas import tpu as pltpu
from jax.experimental.pallas import tpu_sc as plsc

NC = 2
NS = 16
NW = NC * NS
CH = 128


def _sc_gather2(table_pad, idx2, e_rows, pw):
    n_pad, d = table_pad.shape
    nw, nt, ch = idx2.shape
    npt = n_pad // NS
    mesh = plsc.VectorSubcoreMesh(core_axis_name="c", subcore_axis_name="s")

    @functools.partial(
        pl.kernel,
        out_type=(
            jax.ShapeDtypeStruct((e_rows, d), jnp.float32),
            jax.ShapeDtypeStruct((e_rows, d), jnp.float32),
        ),
        mesh=mesh,
        scratch_types=[
            pltpu.VMEM((nt, ch), jnp.int32),
            pltpu.VMEM((2, ch, d), jnp.float32),
            pltpu.VMEM_SHARED((n_pad, d), jnp.float32),
            pltpu.SemaphoreType.DMA,
            pltpu.SemaphoreType.DMA,
        ],
    )
    def k(table_h, idx_h, oi_h, oj_h, idx_v, buf_v, tab_s, sem0, sem1):
        ci = lax.axis_index("c")
        si = lax.axis_index("s")
        wid = si * NC + ci
        base = wid * pw
        pltpu.sync_copy(table_h.at[pl.ds(si * npt, npt)],
                        tab_s.at[pl.ds(si * npt, npt)])
        pltpu.sync_copy(idx_h.at[wid], idx_v)
        plsc.subcore_barrier()

        def off(c):
            return base + lax.min(c * ch, pw - ch)

        pltpu.async_copy(tab_s.at[idx_v.at[0]], buf_v.at[0], sem0)
        pltpu.async_copy(tab_s.at[idx_v.at[1]], buf_v.at[1], sem1)

        def body(t, carry):
            p = t % 2
            c = t // 2

            @pl.when(p == 0)
            def _even():
                pltpu.make_async_copy(tab_s.at[idx_v.at[t]], buf_v.at[0],
                                      sem0).wait()
                pltpu.sync_copy(buf_v.at[0], oi_h.at[pl.ds(off(c), ch)])
                @pl.when(t + 2 < nt)
                def _fire():
                    pltpu.async_copy(tab_s.at[idx_v.at[t + 2]], buf_v.at[0],
                                     sem0)

            @pl.when(p == 1)
            def _odd():
                pltpu.make_async_copy(tab_s.at[idx_v.at[t]], buf_v.at[1],
                                      sem1).wait()
                pltpu.sync_copy(buf_v.at[1], oj_h.at[pl.ds(off(c), ch)])
                @pl.when(t + 2 < nt)
                def _fire():
                    pltpu.async_copy(tab_s.at[idx_v.at[t + 2]], buf_v.at[1],
                                     sem1)

            return carry

        lax.fori_loop(0, nt, body, 0)

    return k(table_pad, idx2)


def _sc_scatter_add(msg, dst3, zeros, n_pad, pw):
    e_rows, d = msg.shape
    nw, cpw, ch = dst3.shape
    npt = n_pad // NS
    mesh = plsc.VectorSubcoreMesh(core_axis_name="c", subcore_axis_name="s")

    @functools.partial(
        pl.kernel,
        out_type=jax.ShapeDtypeStruct((NC, n_pad, d), jnp.float32),
        mesh=mesh,
        scratch_types=[
            pltpu.VMEM((cpw, ch), jnp.int32),
            pltpu.VMEM((2, ch, d), jnp.float32),
            pltpu.VMEM_SHARED((n_pad, d), jnp.float32),
            pltpu.SemaphoreType.DMA,
        ],
    )
    def k(msg_h, dst_h, zeros_h, out_h, idx_v, rows_v, acc_s, sem):
        ci = lax.axis_index("c")
        si = lax.axis_index("s")
        wid = si * NC + ci
        base = wid * pw
        pltpu.sync_copy(zeros_h.at[pl.ds(si * npt, npt)],
                        acc_s.at[pl.ds(si * npt, npt)])
        plsc.subcore_barrier()
        pltpu.sync_copy(dst_h.at[wid], idx_v)

        def off(c):
            return base + lax.min(c * ch, pw - ch)

        pltpu.async_copy(msg_h.at[pl.ds(off(0), ch)], rows_v.at[0], sem)

        def body(c, carry):
            p = c % 2

            @pl.when(c + 1 < cpw)
            def _fire_next():
                pltpu.async_copy(msg_h.at[pl.ds(off(c + 1), ch)],
                                 rows_v.at[1 - p], sem)

            pltpu.make_async_copy(msg_h.at[pl.ds(off(c), ch)], rows_v.at[p],
                                  sem).wait()
            pltpu.sync_copy(rows_v.at[p], acc_s.at[idx_v.at[c]], add=True)
            return carry

        lax.fori_loop(0, cpw, body, 0)
        plsc.subcore_barrier()
        pltpu.sync_copy(acc_s.at[pl.ds(si * npt, npt)],
                        out_h.at[ci, pl.ds(si * npt, npt)])

    return k(msg, dst3, zeros)


def _tc_node_mlp(bi, ang, aux, bj, wbi, wang, waux, wbj, w1t, g1t, woutt,
                 bhg, b1r, g1r, block_rows):
    e_rows, dn = bi.shape
    da = ang.shape[1]
    dx = aux.shape[1]
    h = w1t.shape[0]
    grid = e_rows // block_rows

    def body(bi_r, ang_r, aux_r, bj_r, wbi_r, wang_r, waux_r, wbj_r, w1_r,
             g1_r, wout_r, bhg_r, b1_r, g1b_r, o_r):
        x = jnp.dot(bi_r[...], wbi_r[...], preferred_element_type=jnp.float32)
        x += jnp.dot(ang_r[...], wang_r[...], preferred_element_type=jnp.float32)
        x += jnp.dot(aux_r[...], waux_r[...], preferred_element_type=jnp.float32)
        x += jnp.dot(bj_r[...], wbj_r[...], preferred_element_type=jnp.float32)
        x += bhg_r[...]
        a = x * jax.nn.sigmoid(x)
        ah = a[:, :h]
        ag = a[:, h:]
        h2 = jnp.dot(ah, w1_r[...], preferred_element_type=jnp.float32) + b1_r[...]
        h2 = h2 * jax.nn.sigmoid(h2)
        g2 = jax.nn.sigmoid(
            jnp.dot(ag, g1_r[...], preferred_element_type=jnp.float32) + g1b_r[...])
        m = h2 * g2
        o_r[...] = jnp.dot(m, wout_r[...], preferred_element_type=jnp.float32)

    full = lambda i: (0, 0)
    return pl.pallas_call(
        body,
        grid=(grid,),
        in_specs=[
            pl.BlockSpec((block_rows, dn), lambda i: (i, 0)),
            pl.BlockSpec((block_rows, da), lambda i: (i, 0)),
            pl.BlockSpec((block_rows, dx), lambda i: (i, 0)),
            pl.BlockSpec((block_rows, dn), lambda i: (i, 0)),
            pl.BlockSpec(wbi.shape, full),
            pl.BlockSpec(wang.shape, full),
            pl.BlockSpec(waux.shape, full),
            pl.BlockSpec(wbj.shape, full),
            pl.BlockSpec(w1t.shape, full),
            pl.BlockSpec(g1t.shape, full),
            pl.BlockSpec(woutt.shape, full),
            pl.BlockSpec(bhg.shape, full),
            pl.BlockSpec(b1r.shape, full),
            pl.BlockSpec(g1r.shape, full),
        ],
        out_specs=pl.BlockSpec((block_rows, dn), lambda i: (i, 0)),
        out_shape=jax.ShapeDtypeStruct((e_rows, dn), jnp.float32),
    )(bi, ang, aux, bj, wbi, wang, waux, wbj, w1t, g1t, woutt, bhg, b1r, g1r)


def _tc_edge_mlp(bi, ang, aux, bj, wbi, wang, waux, wbj, w1t, g1t, bhg, b1r,
                 g1r, block_rows):
    e_rows, dn = bi.shape
    da = ang.shape[1]
    dx = aux.shape[1]
    h = w1t.shape[0]
    grid = e_rows // block_rows

    def body(bi_r, ang_r, aux_r, bj_r, wbi_r, wang_r, waux_r, wbj_r, w1_r,
             g1_r, bhg_r, b1_r, g1b_r, o_r):
        x = jnp.dot(bi_r[...], wbi_r[...], preferred_element_type=jnp.float32)
        x += jnp.dot(ang_r[...], wang_r[...], preferred_element_type=jnp.float32)
        x += jnp.dot(aux_r[...], waux_r[...], preferred_element_type=jnp.float32)
        x += jnp.dot(bj_r[...], wbj_r[...], preferred_element_type=jnp.float32)
        x += bhg_r[...]
        a = x * jax.nn.sigmoid(x)
        ah = a[:, :h]
        ag = a[:, h:]
        h2 = jnp.dot(ah, w1_r[...], preferred_element_type=jnp.float32) + b1_r[...]
        h2 = h2 * jax.nn.sigmoid(h2)
        g2 = jax.nn.sigmoid(
            jnp.dot(ag, g1_r[...], preferred_element_type=jnp.float32) + g1b_r[...])
        o_r[...] = ang_r[...] + h2 * g2

    full = lambda i: (0, 0)
    return pl.pallas_call(
        body,
        grid=(grid,),
        in_specs=[
            pl.BlockSpec((block_rows, dn), lambda i: (i, 0)),
            pl.BlockSpec((block_rows, da), lambda i: (i, 0)),
            pl.BlockSpec((block_rows, dx), lambda i: (i, 0)),
            pl.BlockSpec((block_rows, dn), lambda i: (i, 0)),
            pl.BlockSpec(wbi.shape, full),
            pl.BlockSpec(wang.shape, full),
            pl.BlockSpec(waux.shape, full),
            pl.BlockSpec(wbj.shape, full),
            pl.BlockSpec(w1t.shape, full),
            pl.BlockSpec(g1t.shape, full),
            pl.BlockSpec(bhg.shape, full),
            pl.BlockSpec(b1r.shape, full),
            pl.BlockSpec(g1r.shape, full),
        ],
        out_specs=pl.BlockSpec((block_rows, da), lambda i: (i, 0)),
        out_shape=jax.ShapeDtypeStruct((e_rows, da), jnp.float32),
    )(bi, ang, aux, bj, wbi, wang, waux, wbj, w1t, g1t, bhg, b1r, g1r)


def _tc_add_partials(node_feat_pad, parts, block_rows):
    n_pad, d = node_feat_pad.shape
    grid = n_pad // block_rows

    def body(nf_r, p_r, o_r):
        o_r[...] = nf_r[...] + p_r[0] + p_r[1]

    return pl.pallas_call(
        body,
        grid=(grid,),
        in_specs=[
            pl.BlockSpec((block_rows, d), lambda i: (i, 0)),
            pl.BlockSpec((NC, block_rows, d), lambda i: (0, i, 0)),
        ],
        out_specs=pl.BlockSpec((block_rows, d), lambda i: (i, 0)),
        out_shape=jax.ShapeDtypeStruct((n_pad, d), jnp.float32),
    )(node_feat_pad, parts)


def kernel(node_feat, angle_feat, aux_feat, edge_index, W0, b0, W1, b1, G0,
           g0, G1, g1, Wout, We0, be0, We1, be1, Ge0, ge0, Ge1, ge1):
    n, dn = node_feat.shape
    e, da = angle_feat.shape
    dx = aux_feat.shape[1]
    h = W0.shape[0]

    pw = e // NW
    cpw = -(-pw // CH)
    n_grain = 8 * NS
    n_pad = n_grain * (-(-n // n_grain))
    trash = n_pad - 8

    src = edge_index[0]
    dst = edge_index[1]
    offs = jnp.minimum(jnp.arange(cpw, dtype=jnp.int32) * CH, pw - CH)
    pos = (jnp.arange(NW, dtype=jnp.int32)[:, None, None] * pw
           + offs[None, :, None]
           + jnp.arange(CH, dtype=jnp.int32)[None, None, :])
    src3 = src[pos]
    dst3 = dst[pos]
    idx2 = jnp.stack([src3, dst3], axis=2).reshape(NW, 2 * cpw, CH)
    dup = (offs[:, None] + jnp.arange(CH, dtype=jnp.int32)[None, :]
           < jnp.arange(cpw, dtype=jnp.int32)[:, None] * CH)
    dst3_sc = jnp.where(dup[None], trash, dst3)

    node_feat_pad = jnp.pad(node_feat, ((0, n_pad - n), (0, 0)))
    zeros_n = jnp.zeros((n_pad, dn), jnp.float32)

    wg0 = jnp.concatenate([W0.T, G0.T], axis=1)
    wbi0 = wg0[:dn]
    wang0 = wg0[dn:dn + da]
    waux0 = wg0[dn + da:dn + da + dx]
    wbj0 = wg0[dn + da + dx:]
    bhg0 = jnp.concatenate([b0, g0]).reshape(1, 2 * h)
    w1t = W1.T
    g1t = G1.T
    woutt = Wout.T
    b1r = b1.reshape(1, -1)
    g1r = g1.reshape(1, -1)

    wge = jnp.concatenate([We0.T, Ge0.T], axis=1)
    wbie = wge[:dn]
    wange = wge[dn:dn + da]
    wauxe = wge[dn + da:dn + da + dx]
    wbje = wge[dn + da + dx:]
    bhge = jnp.concatenate([be0, ge0]).reshape(1, 2 * h)
    we1t = We1.T
    ge1t = Ge1.T
    be1r = be1.reshape(1, -1)
    ge1r = ge1.reshape(1, -1)

    bi, bj = _sc_gather2(node_feat_pad, idx2, e, pw)
    msgw = _tc_node_mlp(bi, angle_feat, aux_feat, bj, wbi0, wang0, waux0,
                        wbj0, w1t, g1t, woutt, bhg0, b1r, g1r, 1000)
    parts = _sc_scatter_add(msgw, dst3_sc, zeros_n, n_pad, pw)
    new_node_pad = _tc_add_partials(node_feat_pad, parts, n_pad // 16)
    bi2, bj2 = _sc_gather2(new_node_pad, idx2, e, pw)
    new_edge = _tc_edge_mlp(bi2, angle_feat, aux_feat, bj2, wbie, wange,
                            wauxe, wbje, we1t, ge1t, bhge, be1r, ge1r, 1000)
    return new_node_pad[:n], new_edge


# speedup vs baseline: 3.1140x; 1.1298x over previous
import functools

import jax
import jax.numpy as jnp
from jax import lax
from jax.experimental import pallas as pl
from jax.experimental.pallas import tpu as pltpu
from jax.experimental.pallas import tpu_sc as plsc

NC = 2
NS = 16
NW = NC * NS
CH = 128


def _sc_gather2(table_pad, idx2, e_rows, pw):
    n_pad, d = table_pad.shape
    nw, nt, ch = idx2.shape
    npt = n_pad // NS
    mesh = plsc.VectorSubcoreMesh(core_axis_name="c", subcore_axis_name="s")

    @functools.partial(
        pl.kernel,
        out_type=(
            jax.ShapeDtypeStruct((e_rows, d), jnp.float32),
            jax.ShapeDtypeStruct((e_rows, d), jnp.float32),
        ),
        mesh=mesh,
        scratch_types=[
            pltpu.VMEM((nt, ch), jnp.int32),
            pltpu.VMEM((2, ch, d), jnp.float32),
            pltpu.VMEM_SHARED((n_pad, d), jnp.float32),
            pltpu.SemaphoreType.DMA,
            pltpu.SemaphoreType.DMA,
        ],
    )
    def k(table_h, idx_h, oi_h, oj_h, idx_v, buf_v, tab_s, sem0, sem1):
        ci = lax.axis_index("c")
        si = lax.axis_index("s")
        wid = si * NC + ci
        base = wid * pw
        pltpu.sync_copy(table_h.at[pl.ds(si * npt, npt)],
                        tab_s.at[pl.ds(si * npt, npt)])
        pltpu.sync_copy(idx_h.at[wid], idx_v)
        plsc.subcore_barrier()

        def off(c):
            return base + lax.min(c * ch, pw - ch)

        pltpu.async_copy(tab_s.at[idx_v.at[0]], buf_v.at[0], sem0)
        pltpu.async_copy(tab_s.at[idx_v.at[1]], buf_v.at[1], sem1)

        def body(t, carry):
            p = t % 2
            c = t // 2

            @pl.when(p == 0)
            def _even():
                pltpu.make_async_copy(tab_s.at[idx_v.at[t]], buf_v.at[0],
                                      sem0).wait()
                pltpu.sync_copy(buf_v.at[0], oi_h.at[pl.ds(off(c), ch)])
                @pl.when(t + 2 < nt)
                def _fire():
                    pltpu.async_copy(tab_s.at[idx_v.at[t + 2]], buf_v.at[0],
                                     sem0)

            @pl.when(p == 1)
            def _odd():
                pltpu.make_async_copy(tab_s.at[idx_v.at[t]], buf_v.at[1],
                                      sem1).wait()
                pltpu.sync_copy(buf_v.at[1], oj_h.at[pl.ds(off(c), ch)])
                @pl.when(t + 2 < nt)
                def _fire():
                    pltpu.async_copy(tab_s.at[idx_v.at[t + 2]], buf_v.at[1],
                                     sem1)

            return carry

        lax.fori_loop(0, nt, body, 0)

    return k(table_pad, idx2)


def _sc_scatter_add(msg, dst3, zeros, n_pad, pw):
    e_rows, d = msg.shape
    nw, cpw, ch = dst3.shape
    npt = n_pad // NS
    mesh = plsc.VectorSubcoreMesh(core_axis_name="c", subcore_axis_name="s")

    @functools.partial(
        pl.kernel,
        out_type=jax.ShapeDtypeStruct((NC, n_pad, d), jnp.float32),
        mesh=mesh,
        scratch_types=[
            pltpu.VMEM((cpw, ch), jnp.int32),
            pltpu.VMEM((2, ch, d), jnp.float32),
            pltpu.VMEM_SHARED((n_pad, d), jnp.float32),
            pltpu.SemaphoreType.DMA,
        ],
    )
    def k(msg_h, dst_h, zeros_h, out_h, idx_v, rows_v, acc_s, sem):
        ci = lax.axis_index("c")
        si = lax.axis_index("s")
        wid = si * NC + ci
        base = wid * pw
        pltpu.sync_copy(zeros_h.at[pl.ds(si * npt, npt)],
                        acc_s.at[pl.ds(si * npt, npt)])
        plsc.subcore_barrier()
        pltpu.sync_copy(dst_h.at[wid], idx_v)

        def off(c):
            return base + lax.min(c * ch, pw - ch)

        pltpu.async_copy(msg_h.at[pl.ds(off(0), ch)], rows_v.at[0], sem)

        def body(c, carry):
            p = c % 2

            @pl.when(c + 1 < cpw)
            def _fire_next():
                pltpu.async_copy(msg_h.at[pl.ds(off(c + 1), ch)],
                                 rows_v.at[1 - p], sem)

            pltpu.make_async_copy(msg_h.at[pl.ds(off(c), ch)], rows_v.at[p],
                                  sem).wait()
            pltpu.sync_copy(rows_v.at[p], acc_s.at[idx_v.at[c]], add=True)
            return carry

        lax.fori_loop(0, cpw, body, 0)
        plsc.subcore_barrier()
        pltpu.sync_copy(acc_s.at[pl.ds(si * npt, npt)],
                        out_h.at[ci, pl.ds(si * npt, npt)])

    return k(msg, dst3, zeros)


def _tc_node_mlp(bi, ang, aux, bj, wg0, w1t, g1t, woutt, bhg, b1r, g1r,
                 block_rows):
    e_rows, dn = bi.shape
    da = ang.shape[1]
    dx = aux.shape[1]
    h = w1t.shape[0]
    grid = e_rows // block_rows

    bf = jnp.bfloat16

    def body(bi_r, ang_r, aux_r, bj_r, wg0_r, w1_r, g1_r, wout_r, bhg_r,
             b1_r, g1b_r, o_r):
        xc = jnp.concatenate(
            [bi_r[...].astype(bf), ang_r[...].astype(bf),
             aux_r[...].astype(bf), bj_r[...].astype(bf)], axis=1)
        x = jnp.dot(xc, wg0_r[...], preferred_element_type=jnp.float32)
        x += bhg_r[...]
        a = x * jax.nn.sigmoid(x)
        ah = a[:, :h].astype(bf)
        ag = a[:, h:].astype(bf)
        h2 = jnp.dot(ah, w1_r[...], preferred_element_type=jnp.float32) + b1_r[...]
        h2 = h2 * jax.nn.sigmoid(h2)
        g2 = jax.nn.sigmoid(
            jnp.dot(ag, g1_r[...], preferred_element_type=jnp.float32) + g1b_r[...])
        m = (h2 * g2).astype(bf)
        o_r[...] = jnp.dot(m, wout_r[...], preferred_element_type=jnp.float32)

    full = lambda i: (0, 0)
    return pl.pallas_call(
        body,
        grid=(grid,),
        in_specs=[
            pl.BlockSpec((block_rows, dn), lambda i: (i, 0)),
            pl.BlockSpec((block_rows, da), lambda i: (i, 0)),
            pl.BlockSpec((block_rows, dx), lambda i: (i, 0)),
            pl.BlockSpec((block_rows, dn), lambda i: (i, 0)),
            pl.BlockSpec(wg0.shape, full),
            pl.BlockSpec(w1t.shape, full),
            pl.BlockSpec(g1t.shape, full),
            pl.BlockSpec(woutt.shape, full),
            pl.BlockSpec(bhg.shape, full),
            pl.BlockSpec(b1r.shape, full),
            pl.BlockSpec(g1r.shape, full),
        ],
        out_specs=pl.BlockSpec((block_rows, dn), lambda i: (i, 0)),
        out_shape=jax.ShapeDtypeStruct((e_rows, dn), jnp.float32),
    )(bi, ang, aux, bj, wg0, w1t, g1t, woutt, bhg, b1r, g1r)


def _tc_edge_mlp(bi, ang, aux, bj, wge, w1t, g1t, bhg, b1r, g1r, block_rows):
    e_rows, dn = bi.shape
    da = ang.shape[1]
    dx = aux.shape[1]
    h = w1t.shape[0]
    grid = e_rows // block_rows

    bf = jnp.bfloat16

    def body(bi_r, ang_r, aux_r, bj_r, wge_r, w1_r, g1_r, bhg_r, b1_r,
             g1b_r, o_r):
        xc = jnp.concatenate(
            [bi_r[...].astype(bf), ang_r[...].astype(bf),
             aux_r[...].astype(bf), bj_r[...].astype(bf)], axis=1)
        x = jnp.dot(xc, wge_r[...], preferred_element_type=jnp.float32)
        x += bhg_r[...]
        a = x * jax.nn.sigmoid(x)
        ah = a[:, :h].astype(bf)
        ag = a[:, h:].astype(bf)
        h2 = jnp.dot(ah, w1_r[...], preferred_element_type=jnp.float32) + b1_r[...]
        h2 = h2 * jax.nn.sigmoid(h2)
        g2 = jax.nn.sigmoid(
            jnp.dot(ag, g1_r[...], preferred_element_type=jnp.float32) + g1b_r[...])
        o_r[...] = ang_r[...] + h2 * g2

    full = lambda i: (0, 0)
    return pl.pallas_call(
        body,
        grid=(grid,),
        in_specs=[
            pl.BlockSpec((block_rows, dn), lambda i: (i, 0)),
            pl.BlockSpec((block_rows, da), lambda i: (i, 0)),
            pl.BlockSpec((block_rows, dx), lambda i: (i, 0)),
            pl.BlockSpec((block_rows, dn), lambda i: (i, 0)),
            pl.BlockSpec(wge.shape, full),
            pl.BlockSpec(w1t.shape, full),
            pl.BlockSpec(g1t.shape, full),
            pl.BlockSpec(bhg.shape, full),
            pl.BlockSpec(b1r.shape, full),
            pl.BlockSpec(g1r.shape, full),
        ],
        out_specs=pl.BlockSpec((block_rows, da), lambda i: (i, 0)),
        out_shape=jax.ShapeDtypeStruct((e_rows, da), jnp.float32),
    )(bi, ang, aux, bj, wge, w1t, g1t, bhg, b1r, g1r)


def _tc_add_partials(node_feat_pad, parts, block_rows):
    n_pad, d = node_feat_pad.shape
    grid = n_pad // block_rows

    def body(nf_r, p_r, o_r):
        o_r[...] = nf_r[...] + p_r[0] + p_r[1]

    return pl.pallas_call(
        body,
        grid=(grid,),
        in_specs=[
            pl.BlockSpec((block_rows, d), lambda i: (i, 0)),
            pl.BlockSpec((NC, block_rows, d), lambda i: (0, i, 0)),
        ],
        out_specs=pl.BlockSpec((block_rows, d), lambda i: (i, 0)),
        out_shape=jax.ShapeDtypeStruct((n_pad, d), jnp.float32),
    )(node_feat_pad, parts)


def kernel(node_feat, angle_feat, aux_feat, edge_index, W0, b0, W1, b1, G0,
           g0, G1, g1, Wout, We0, be0, We1, be1, Ge0, ge0, Ge1, ge1):
    n, dn = node_feat.shape
    e, da = angle_feat.shape
    dx = aux_feat.shape[1]
    h = W0.shape[0]

    pw = e // NW
    cpw = -(-pw // CH)
    n_grain = 8 * NS
    n_pad = n_grain * (-(-n // n_grain))
    trash = n_pad - 8

    src = edge_index[0]
    dst = edge_index[1]

    def chunks3(v):
        vw = v.reshape(NW, pw)
        main = vw[:, :CH * (cpw - 1)].reshape(NW, cpw - 1, CH)
        tail = vw[:, pw - CH:].reshape(NW, 1, CH)
        return jnp.concatenate([main, tail], axis=1)

    src3 = chunks3(src)
    dst3 = chunks3(dst)
    idx2 = jnp.stack([src3, dst3], axis=2).reshape(NW, 2 * cpw, CH)
    offs = jnp.minimum(jnp.arange(cpw, dtype=jnp.int32) * CH, pw - CH)
    dup = (offs[:, None] + jnp.arange(CH, dtype=jnp.int32)[None, :]
           < jnp.arange(cpw, dtype=jnp.int32)[:, None] * CH)
    dst3_sc = jnp.where(dup[None], trash, dst3)

    node_feat_pad = jnp.pad(node_feat, ((0, n_pad - n), (0, 0)))
    zeros_n = jnp.zeros((n_pad, dn), jnp.float32)

    bf = jnp.bfloat16
    wg0 = jnp.concatenate([W0.T, G0.T], axis=1).astype(bf)
    bhg0 = jnp.concatenate([b0, g0]).reshape(1, 2 * h)
    w1t = W1.T.astype(bf)
    g1t = G1.T.astype(bf)
    woutt = Wout.T.astype(bf)
    b1r = b1.reshape(1, -1)
    g1r = g1.reshape(1, -1)

    wge = jnp.concatenate([We0.T, Ge0.T], axis=1).astype(bf)
    bhge = jnp.concatenate([be0, ge0]).reshape(1, 2 * h)
    we1t = We1.T.astype(bf)
    ge1t = Ge1.T.astype(bf)
    be1r = be1.reshape(1, -1)
    ge1r = ge1.reshape(1, -1)

    bi, bj = _sc_gather2(node_feat_pad, idx2, e, pw)
    msgw = _tc_node_mlp(bi, angle_feat, aux_feat, bj, wg0, w1t, g1t, woutt,
                        bhg0, b1r, g1r, 1000)
    parts = _sc_scatter_add(msgw, dst3_sc, zeros_n, n_pad, pw)
    new_node_pad = _tc_add_partials(node_feat_pad, parts, n_pad // 16)
    bi2, bj2 = _sc_gather2(new_node_pad, idx2, e, pw)
    new_edge = _tc_edge_mlp(bi2, angle_feat, aux_feat, bj2, wge, we1t, ge1t,
                            bhge, be1r, ge1r, 1000)
    return new_node_pad[:n], new_edge


# speedup vs baseline: 3.4893x; 1.1205x over previous
import functools

import jax
import jax.numpy as jnp
from jax import lax
from jax.experimental import pallas as pl
from jax.experimental.pallas import tpu as pltpu
from jax.experimental.pallas import tpu_sc as plsc

NC = 2
NS = 16
NW = NC * NS
CH = 128
BF = jnp.bfloat16


def _sc_gather2(table_pad, idx2, e_rows, nch):
    n_pad, d = table_pad.shape
    nw, nt, ch = idx2.shape
    npt = n_pad // NS
    mesh = plsc.VectorSubcoreMesh(core_axis_name="c", subcore_axis_name="s")

    @functools.partial(
        pl.kernel,
        out_type=(
            jax.ShapeDtypeStruct((e_rows, d), jnp.float32),
            jax.ShapeDtypeStruct((e_rows, d), jnp.float32),
        ),
        mesh=mesh,
        scratch_types=[
            pltpu.VMEM((nt, ch), jnp.int32),
            pltpu.VMEM((2, ch, d), jnp.float32),
            pltpu.VMEM_SHARED((n_pad, d), jnp.float32),
            pltpu.SemaphoreType.DMA,
            pltpu.SemaphoreType.DMA,
        ],
    )
    def k(table_h, idx_h, oi_h, oj_h, idx_v, buf_v, tab_s, sem0, sem1):
        ci = lax.axis_index("c")
        si = lax.axis_index("s")
        wid = si * NC + ci
        pltpu.sync_copy(table_h.at[pl.ds(si * npt, npt)],
                        tab_s.at[pl.ds(si * npt, npt)])
        pltpu.sync_copy(idx_h.at[wid], idx_v)
        plsc.subcore_barrier()

        nt_w = 2 * ((nch - wid + NW - 1) // NW)

        def off(k_):
            return (wid + k_ * NW) * ch

        pltpu.async_copy(tab_s.at[idx_v.at[0]], buf_v.at[0], sem0)
        pltpu.async_copy(tab_s.at[idx_v.at[1]], buf_v.at[1], sem1)

        def body(t, carry):
            p = t % 2
            c = t // 2

            @pl.when(p == 0)
            def _even():
                pltpu.make_async_copy(tab_s.at[idx_v.at[t]], buf_v.at[0],
                                      sem0).wait()
                pltpu.sync_copy(buf_v.at[0], oi_h.at[pl.ds(off(c), ch)])
                @pl.when(t + 2 < nt_w)
                def _fire():
                    pltpu.async_copy(tab_s.at[idx_v.at[t + 2]], buf_v.at[0],
                                     sem0)

            @pl.when(p == 1)
            def _odd():
                pltpu.make_async_copy(tab_s.at[idx_v.at[t]], buf_v.at[1],
                                      sem1).wait()
                pltpu.sync_copy(buf_v.at[1], oj_h.at[pl.ds(off(c), ch)])
                @pl.when(t + 2 < nt_w)
                def _fire():
                    pltpu.async_copy(tab_s.at[idx_v.at[t + 2]], buf_v.at[1],
                                     sem1)

            return carry

        lax.fori_loop(0, nt_w, body, 0)

    return k(table_pad, idx2)


def _sc_scatter_add(msg, dst3, zeros, n_pad, nch):
    e_rows, d = msg.shape
    nw, kmax, ch = dst3.shape
    npt = n_pad // NS
    mesh = plsc.VectorSubcoreMesh(core_axis_name="c", subcore_axis_name="s")

    @functools.partial(
        pl.kernel,
        out_type=jax.ShapeDtypeStruct((NC, n_pad, d), jnp.float32),
        mesh=mesh,
        scratch_types=[
            pltpu.VMEM((kmax, ch), jnp.int32),
            pltpu.VMEM((2, ch, d), jnp.float32),
            pltpu.VMEM_SHARED((n_pad, d), jnp.float32),
            pltpu.SemaphoreType.DMA,
        ],
    )
    def k(msg_h, dst_h, zeros_h, out_h, idx_v, rows_v, acc_s, sem):
        ci = lax.axis_index("c")
        si = lax.axis_index("s")
        wid = si * NC + ci
        pltpu.sync_copy(zeros_h.at[pl.ds(si * npt, npt)],
                        acc_s.at[pl.ds(si * npt, npt)])
        plsc.subcore_barrier()
        pltpu.sync_copy(dst_h.at[wid], idx_v)

        nk_w = (nch - wid + NW - 1) // NW

        def off(k_):
            return (wid + k_ * NW) * ch

        pltpu.async_copy(msg_h.at[pl.ds(off(0), ch)], rows_v.at[0], sem)

        def body(c, carry):
            p = c % 2

            @pl.when(c + 1 < nk_w)
            def _fire_next():
                pltpu.async_copy(msg_h.at[pl.ds(off(c + 1), ch)],
                                 rows_v.at[1 - p], sem)

            pltpu.make_async_copy(msg_h.at[pl.ds(off(c), ch)], rows_v.at[p],
                                  sem).wait()
            pltpu.sync_copy(rows_v.at[p], acc_s.at[idx_v.at[c]], add=True)
            return carry

        lax.fori_loop(0, nk_w, body, 0)
        plsc.subcore_barrier()
        pltpu.sync_copy(acc_s.at[pl.ds(si * npt, npt)],
                        out_h.at[ci, pl.ds(si * npt, npt)])

    return k(msg, dst3, zeros)


def _tc_node_mlp(bi, ang, aux, bj, wg0, w1t, g1t, woutt, bhg, b1r, g1r,
                 block_rows):
    e_rows, dn = bi.shape
    da = ang.shape[1]
    dx = aux.shape[1]
    h = w1t.shape[0]
    grid = e_rows // block_rows

    def body(bi_r, ang_r, aux_r, bj_r, wg0_r, w1_r, g1_r, wout_r, bhg_r,
             b1_r, g1b_r, o_r):
        xc = jnp.concatenate(
            [bi_r[...].astype(BF), ang_r[...], aux_r[...],
             bj_r[...].astype(BF)], axis=1)
        x = jnp.dot(xc, wg0_r[...], preferred_element_type=jnp.float32)
        x += bhg_r[...]
        a = x * jax.nn.sigmoid(x)
        ah = a[:, :h].astype(BF)
        ag = a[:, h:].astype(BF)
        h2 = jnp.dot(ah, w1_r[...], preferred_element_type=jnp.float32) + b1_r[...]
        h2 = h2 * jax.nn.sigmoid(h2)
        g2 = jax.nn.sigmoid(
            jnp.dot(ag, g1_r[...], preferred_element_type=jnp.float32) + g1b_r[...])
        m = (h2 * g2).astype(BF)
        o_r[...] = jnp.dot(m, wout_r[...], preferred_element_type=jnp.float32)

    full = lambda i: (0, 0)
    return pl.pallas_call(
        body,
        grid=(grid,),
        in_specs=[
            pl.BlockSpec((block_rows, dn), lambda i: (i, 0)),
            pl.BlockSpec((block_rows, da), lambda i: (i, 0)),
            pl.BlockSpec((block_rows, dx), lambda i: (i, 0)),
            pl.BlockSpec((block_rows, dn), lambda i: (i, 0)),
            pl.BlockSpec(wg0.shape, full),
            pl.BlockSpec(w1t.shape, full),
            pl.BlockSpec(g1t.shape, full),
            pl.BlockSpec(woutt.shape, full),
            pl.BlockSpec(bhg.shape, full),
            pl.BlockSpec(b1r.shape, full),
            pl.BlockSpec(g1r.shape, full),
        ],
        out_specs=pl.BlockSpec((block_rows, dn), lambda i: (i, 0)),
        out_shape=jax.ShapeDtypeStruct((e_rows, dn), jnp.float32),
    )(bi, ang, aux, bj, wg0, w1t, g1t, woutt, bhg, b1r, g1r)


def _tc_edge_mlp(bi, ang, aux, bj, wge, w1t, g1t, bhg, b1r, g1r, block_rows):
    e_rows, dn = bi.shape
    da = ang.shape[1]
    dx = aux.shape[1]
    h = w1t.shape[0]
    grid = e_rows // block_rows

    def body(bi_r, ang_r, aux_r, bj_r, wge_r, w1_r, g1_r, bhg_r, b1_r,
             g1b_r, o_r):
        xc = jnp.concatenate(
            [bi_r[...].astype(BF), ang_r[...].astype(BF), aux_r[...],
             bj_r[...].astype(BF)], axis=1)
        x = jnp.dot(xc, wge_r[...], preferred_element_type=jnp.float32)
        x += bhg_r[...]
        a = x * jax.nn.sigmoid(x)
        ah = a[:, :h].astype(BF)
        ag = a[:, h:].astype(BF)
        h2 = jnp.dot(ah, w1_r[...], preferred_element_type=jnp.float32) + b1_r[...]
        h2 = h2 * jax.nn.sigmoid(h2)
        g2 = jax.nn.sigmoid(
            jnp.dot(ag, g1_r[...], preferred_element_type=jnp.float32) + g1b_r[...])
        o_r[...] = ang_r[...] + h2 * g2

    full = lambda i: (0, 0)
    return pl.pallas_call(
        body,
        grid=(grid,),
        in_specs=[
            pl.BlockSpec((block_rows, dn), lambda i: (i, 0)),
            pl.BlockSpec((block_rows, da), lambda i: (i, 0)),
            pl.BlockSpec((block_rows, dx), lambda i: (i, 0)),
            pl.BlockSpec((block_rows, dn), lambda i: (i, 0)),
            pl.BlockSpec(wge.shape, full),
            pl.BlockSpec(w1t.shape, full),
            pl.BlockSpec(g1t.shape, full),
            pl.BlockSpec(bhg.shape, full),
            pl.BlockSpec(b1r.shape, full),
            pl.BlockSpec(g1r.shape, full),
        ],
        out_specs=pl.BlockSpec((block_rows, da), lambda i: (i, 0)),
        out_shape=jax.ShapeDtypeStruct((e_rows, da), jnp.float32),
    )(bi, ang, aux, bj, wge, w1t, g1t, bhg, b1r, g1r)


def _tc_add_partials(node_feat, parts, n_pad, block_rows):
    n, d = node_feat.shape
    grid = n_pad // block_rows

    def body(nf_r, p_r, o_r, op_r):
        o = nf_r[...] + p_r[0] + p_r[1]
        o_r[...] = o
        op_r[...] = o

    return pl.pallas_call(
        body,
        grid=(grid,),
        in_specs=[
            pl.BlockSpec((block_rows, d), lambda i: (i, 0)),
            pl.BlockSpec((NC, block_rows, d), lambda i: (0, i, 0)),
        ],
        out_specs=[
            pl.BlockSpec((block_rows, d), lambda i: (i, 0)),
            pl.BlockSpec((block_rows, d), lambda i: (i, 0)),
        ],
        out_shape=[
            jax.ShapeDtypeStruct((n, d), jnp.float32),
            jax.ShapeDtypeStruct((n_pad, d), jnp.float32),
        ],
    )(node_feat, parts)


def kernel(node_feat, angle_feat, aux_feat, edge_index, W0, b0, W1, b1, G0,
           g0, G1, g1, Wout, We0, be0, We1, be1, Ge0, ge0, Ge1, ge1):
    n, dn = node_feat.shape
    e, da = angle_feat.shape
    dx = aux_feat.shape[1]
    h = W0.shape[0]

    nch = e // CH
    kmax = -(-nch // NW)
    n_grain = 8 * NS
    n_pad = n_grain * (-(-n // n_grain))

    src = edge_index[0]
    dst = edge_index[1]
    padc = kmax * NW - nch

    def chunks3(v):
        vp = jnp.pad(v, (0, padc * CH))
        return vp.reshape(kmax, NW, CH).transpose(1, 0, 2)

    src3 = chunks3(src)
    dst3 = chunks3(dst)
    idx2 = jnp.stack([src3, dst3], axis=2).reshape(NW, 2 * kmax, CH)

    node_pad = jnp.pad(node_feat, ((0, n_pad - n), (0, 0)))
    zeros_n = jnp.zeros((n_pad, dn), jnp.float32)
    ang16 = angle_feat.astype(BF)
    aux16 = aux_feat.astype(BF)

    wg0 = jnp.concatenate([W0.T, G0.T], axis=1).astype(BF)
    bhg0 = jnp.concatenate([b0, g0]).reshape(1, 2 * h)
    w1t = W1.T.astype(BF)
    g1t = G1.T.astype(BF)
    woutt = Wout.T.astype(BF)
    b1r = b1.reshape(1, -1)
    g1r = g1.reshape(1, -1)

    wge = jnp.concatenate([We0.T, Ge0.T], axis=1).astype(BF)
    bhge = jnp.concatenate([be0, ge0]).reshape(1, 2 * h)
    we1t = We1.T.astype(BF)
    ge1t = Ge1.T.astype(BF)
    be1r = be1.reshape(1, -1)
    ge1r = ge1.reshape(1, -1)

    bi, bj = _sc_gather2(node_pad, idx2, e, nch)
    msgw = _tc_node_mlp(bi, ang16, aux16, bj, wg0, w1t, g1t, woutt, bhg0,
                        b1r, g1r, 2000)
    parts = _sc_scatter_add(msgw, dst3, zeros_n, n_pad, nch)
    new_node, new_node_pad = _tc_add_partials(node_feat, parts, n_pad,
                                             n_pad // 16)
    bi2, bj2 = _sc_gather2(new_node_pad, idx2, e, nch)
    new_edge = _tc_edge_mlp(bi2, angle_feat, aux16, bj2, wge, we1t, ge1t,
                            bhge, be1r, ge1r, 2000)
    return new_node, new_edge


# speedup vs baseline: 3.8232x; 1.0957x over previous
import functools

import jax
import jax.numpy as jnp
from jax import lax
from jax.experimental import pallas as pl
from jax.experimental.pallas import tpu as pltpu
from jax.experimental.pallas import tpu_sc as plsc

NC = 2
NS = 16
NW = NC * NS
CH = 128
BF = jnp.bfloat16


def _pack16(x):
    lo = lax.bitcast_convert_type(x[:, :64].astype(BF), jnp.uint16)
    hi = lax.bitcast_convert_type(x[:, 64:].astype(BF), jnp.uint16)
    return (lo.astype(jnp.int32) | (hi.astype(jnp.int32) << 16))


def _unpack16(w):
    u = lax.bitcast_convert_type(w, jnp.uint32)
    lo = lax.bitcast_convert_type((u & jnp.uint32(0xFFFF)).astype(jnp.uint16), BF)
    hi = lax.bitcast_convert_type((u >> 16).astype(jnp.uint16), BF)
    return jnp.concatenate([lo, hi], axis=1)


def _sc_gather2(table_pad, idx2, e_rows, nch):
    n_pad, d = table_pad.shape
    nw, nt, ch = idx2.shape
    npt = n_pad // NS
    mesh = plsc.VectorSubcoreMesh(core_axis_name="c", subcore_axis_name="s")

    @functools.partial(
        pl.kernel,
        out_type=(
            jax.ShapeDtypeStruct((e_rows, d), jnp.float32),
            jax.ShapeDtypeStruct((e_rows, d), jnp.float32),
        ),
        mesh=mesh,
        scratch_types=[
            pltpu.VMEM((nt, ch), jnp.int32),
            pltpu.VMEM((2, ch, d), jnp.float32),
            pltpu.VMEM_SHARED((n_pad, d), jnp.float32),
            pltpu.SemaphoreType.DMA,
            pltpu.SemaphoreType.DMA,
        ],
    )
    def k(table_h, idx_h, oi_h, oj_h, idx_v, buf_v, tab_s, sem0, sem1):
        ci = lax.axis_index("c")
        si = lax.axis_index("s")
        wid = si * NC + ci
        pltpu.sync_copy(table_h.at[pl.ds(si * npt, npt)],
                        tab_s.at[pl.ds(si * npt, npt)])
        pltpu.sync_copy(idx_h.at[wid], idx_v)
        plsc.subcore_barrier()

        nt_w = 2 * ((nch - wid + NW - 1) // NW)

        def off(k_):
            return (wid + k_ * NW) * ch

        pltpu.async_copy(tab_s.at[idx_v.at[0]], buf_v.at[0], sem0)
        pltpu.async_copy(tab_s.at[idx_v.at[1]], buf_v.at[1], sem1)

        def body(t, carry):
            p = t % 2
            c = t // 2

            @pl.when(p == 0)
            def _even():
                pltpu.make_async_copy(tab_s.at[idx_v.at[t]], buf_v.at[0],
                                      sem0).wait()
                pltpu.sync_copy(buf_v.at[0], oi_h.at[pl.ds(off(c), ch)])
                @pl.when(t + 2 < nt_w)
                def _fire():
                    pltpu.async_copy(tab_s.at[idx_v.at[t + 2]], buf_v.at[0],
                                     sem0)

            @pl.when(p == 1)
            def _odd():
                pltpu.make_async_copy(tab_s.at[idx_v.at[t]], buf_v.at[1],
                                      sem1).wait()
                pltpu.sync_copy(buf_v.at[1], oj_h.at[pl.ds(off(c), ch)])
                @pl.when(t + 2 < nt_w)
                def _fire():
                    pltpu.async_copy(tab_s.at[idx_v.at[t + 2]], buf_v.at[1],
                                     sem1)

            return carry

        lax.fori_loop(0, nt_w, body, 0)

    return k(table_pad, idx2)


def _sc_scatter_add(msg, dst3, zeros, n_pad, nch):
    e_rows, d = msg.shape
    nw, kmax, ch = dst3.shape
    npt = n_pad // NS
    mesh = plsc.VectorSubcoreMesh(core_axis_name="c", subcore_axis_name="s")

    @functools.partial(
        pl.kernel,
        out_type=jax.ShapeDtypeStruct((NC, n_pad, d), jnp.float32),
        mesh=mesh,
        scratch_types=[
            pltpu.VMEM((kmax, ch), jnp.int32),
            pltpu.VMEM((2, ch, d), jnp.float32),
            pltpu.VMEM_SHARED((n_pad, d), jnp.float32),
            pltpu.SemaphoreType.DMA,
        ],
    )
    def k(msg_h, dst_h, zeros_h, out_h, idx_v, rows_v, acc_s, sem):
        ci = lax.axis_index("c")
        si = lax.axis_index("s")
        wid = si * NC + ci
        pltpu.sync_copy(zeros_h.at[pl.ds(si * npt, npt)],
                        acc_s.at[pl.ds(si * npt, npt)])
        plsc.subcore_barrier()
        pltpu.sync_copy(dst_h.at[wid], idx_v)

        nk_w = (nch - wid + NW - 1) // NW

        def off(k_):
            return (wid + k_ * NW) * ch

        pltpu.async_copy(msg_h.at[pl.ds(off(0), ch)], rows_v.at[0], sem)

        def body(c, carry):
            p = c % 2

            @pl.when(c + 1 < nk_w)
            def _fire_next():
                pltpu.async_copy(msg_h.at[pl.ds(off(c + 1), ch)],
                                 rows_v.at[1 - p], sem)

            pltpu.make_async_copy(msg_h.at[pl.ds(off(c), ch)], rows_v.at[p],
                                  sem).wait()
            pltpu.sync_copy(rows_v.at[p], acc_s.at[idx_v.at[c]], add=True)
            return carry

        lax.fori_loop(0, nk_w, body, 0)
        plsc.subcore_barrier()
        pltpu.sync_copy(acc_s.at[pl.ds(si * npt, npt)],
                        out_h.at[ci, pl.ds(si * npt, npt)])

    return k(msg, dst3, zeros)


def _tc_node_mlp(bi, angt, auxt, bj, wg0, w1t, g1t, woutt, bhg, b1r, g1r,
                 block_rows):
    e_rows, dn = bi.shape
    da = angt.shape[0]
    dx = auxt.shape[0]
    h = w1t.shape[0]
    grid = e_rows // block_rows

    def body(bi_r, angt_r, auxt_r, bj_r, wg0_r, w1_r, g1_r, wout_r, bhg_r,
             b1_r, g1b_r, o_r):
        xc = jnp.concatenate(
            [bi_r[...].astype(BF), angt_r[...].T, auxt_r[...].T,
             bj_r[...].astype(BF)], axis=1)
        x = jnp.dot(xc, wg0_r[...], preferred_element_type=jnp.float32)
        x += bhg_r[...]
        a = x * jax.nn.sigmoid(x)
        ah = a[:, :h].astype(BF)
        ag = a[:, h:].astype(BF)
        h2 = jnp.dot(ah, w1_r[...], preferred_element_type=jnp.float32) + b1_r[...]
        h2 = h2 * jax.nn.sigmoid(h2)
        g2 = jax.nn.sigmoid(
            jnp.dot(ag, g1_r[...], preferred_element_type=jnp.float32) + g1b_r[...])
        m = (h2 * g2).astype(BF)
        o_r[...] = jnp.dot(m, wout_r[...], preferred_element_type=jnp.float32)

    full = lambda i: (0, 0)
    return pl.pallas_call(
        body,
        grid=(grid,),
        in_specs=[
            pl.BlockSpec((block_rows, dn), lambda i: (i, 0)),
            pl.BlockSpec((da, block_rows), lambda i: (0, i)),
            pl.BlockSpec((dx, block_rows), lambda i: (0, i)),
            pl.BlockSpec((block_rows, dn), lambda i: (i, 0)),
            pl.BlockSpec(wg0.shape, full),
            pl.BlockSpec(w1t.shape, full),
            pl.BlockSpec(g1t.shape, full),
            pl.BlockSpec(woutt.shape, full),
            pl.BlockSpec(bhg.shape, full),
            pl.BlockSpec(b1r.shape, full),
            pl.BlockSpec(g1r.shape, full),
        ],
        out_specs=pl.BlockSpec((block_rows, dn), lambda i: (i, 0)),
        out_shape=jax.ShapeDtypeStruct((e_rows, dn), jnp.float32),
    )(bi, angt, auxt, bj, wg0, w1t, g1t, woutt, bhg, b1r, g1r)


def _tc_edge_mlp(bi, angt, auxt, bj, wge, w1t, g1t, bhg, b1r, g1r,
                 block_rows):
    e_rows, dn = bi.shape
    da = angt.shape[0]
    dx = auxt.shape[0]
    h = w1t.shape[0]
    grid = e_rows // block_rows

    def body(bi_r, angt_r, auxt_r, bj_r, wge_r, w1_r, g1_r, bhg_r, b1_r,
             g1b_r, o_r):
        ang = angt_r[...].T
        xc = jnp.concatenate(
            [bi_r[...].astype(BF), ang.astype(BF), auxt_r[...].T,
             bj_r[...].astype(BF)], axis=1)
        x = jnp.dot(xc, wge_r[...], preferred_element_type=jnp.float32)
        x += bhg_r[...]
        a = x * jax.nn.sigmoid(x)
        ah = a[:, :h].astype(BF)
        ag = a[:, h:].astype(BF)
        h2 = jnp.dot(ah, w1_r[...], preferred_element_type=jnp.float32) + b1_r[...]
        h2 = h2 * jax.nn.sigmoid(h2)
        g2 = jax.nn.sigmoid(
            jnp.dot(ag, g1_r[...], preferred_element_type=jnp.float32) + g1b_r[...])
        o_r[...] = (ang + h2 * g2).T

    full = lambda i: (0, 0)
    return pl.pallas_call(
        body,
        grid=(grid,),
        in_specs=[
            pl.BlockSpec((block_rows, dn), lambda i: (i, 0)),
            pl.BlockSpec((da, block_rows), lambda i: (0, i)),
            pl.BlockSpec((dx, block_rows), lambda i: (0, i)),
            pl.BlockSpec((block_rows, dn), lambda i: (i, 0)),
            pl.BlockSpec(wge.shape, full),
            pl.BlockSpec(w1t.shape, full),
            pl.BlockSpec(g1t.shape, full),
            pl.BlockSpec(bhg.shape, full),
            pl.BlockSpec(b1r.shape, full),
            pl.BlockSpec(g1r.shape, full),
        ],
        out_specs=pl.BlockSpec((da, block_rows), lambda i: (0, i)),
        out_shape=jax.ShapeDtypeStruct((da, e_rows), jnp.float32),
    )(bi, angt, auxt, bj, wge, w1t, g1t, bhg, b1r, g1r)


def _tc_add_partials(node_feat, parts, n_pad, block_rows):
    n, d = node_feat.shape
    grid = n_pad // block_rows

    def body(nf_r, p_r, o_r, op_r):
        o = nf_r[...] + p_r[0] + p_r[1]
        o_r[...] = o
        op_r[...] = o

    return pl.pallas_call(
        body,
        grid=(grid,),
        in_specs=[
            pl.BlockSpec((block_rows, d), lambda i: (i, 0)),
            pl.BlockSpec((NC, block_rows, d), lambda i: (0, i, 0)),
        ],
        out_specs=[
            pl.BlockSpec((block_rows, d), lambda i: (i, 0)),
            pl.BlockSpec((block_rows, d), lambda i: (i, 0)),
        ],
        out_shape=[
            jax.ShapeDtypeStruct((n, d), jnp.float32),
            jax.ShapeDtypeStruct((n_pad, d), jnp.float32),
        ],
    )(node_feat, parts)


def kernel(node_feat, angle_feat, aux_feat, edge_index, W0, b0, W1, b1, G0,
           g0, G1, g1, Wout, We0, be0, We1, be1, Ge0, ge0, Ge1, ge1):
    n, dn = node_feat.shape
    e, da = angle_feat.shape
    dx = aux_feat.shape[1]
    h = W0.shape[0]

    nch = e // CH
    kmax = -(-nch // NW)
    n_grain = 8 * NS
    n_pad = n_grain * (-(-n // n_grain))

    src = edge_index[0]
    dst = edge_index[1]
    padc = kmax * NW - nch

    def chunks3(v):
        vp = jnp.pad(v, (0, padc * CH))
        return vp.reshape(kmax, NW, CH).transpose(1, 0, 2)

    src3 = chunks3(src)
    dst3 = chunks3(dst)
    idx2 = jnp.stack([src3, dst3], axis=2).reshape(NW, 2 * kmax, CH)

    node_pad = jnp.pad(node_feat, ((0, n_pad - n), (0, 0)))
    zeros_n = jnp.zeros((n_pad, dn), jnp.float32)
    angt = angle_feat.T
    auxt = aux_feat.T
    angt16 = angt.astype(BF)
    auxt16 = auxt.astype(BF)

    wg0 = jnp.concatenate([W0.T, G0.T], axis=1).astype(BF)
    bhg0 = jnp.concatenate([b0, g0]).reshape(1, 2 * h)
    w1t = W1.T.astype(BF)
    g1t = G1.T.astype(BF)
    woutt = Wout.T.astype(BF)
    b1r = b1.reshape(1, -1)
    g1r = g1.reshape(1, -1)

    wge = jnp.concatenate([We0.T, Ge0.T], axis=1).astype(BF)
    bhge = jnp.concatenate([be0, ge0]).reshape(1, 2 * h)
    we1t = We1.T.astype(BF)
    ge1t = Ge1.T.astype(BF)
    be1r = be1.reshape(1, -1)
    ge1r = ge1.reshape(1, -1)

    bi, bj = _sc_gather2(node_pad, idx2, e, nch)
    msgw = _tc_node_mlp(bi, angt16, auxt16, bj, wg0, w1t, g1t, woutt, bhg0,
                        b1r, g1r, 3200)
    parts = _sc_scatter_add(msgw, dst3, zeros_n, n_pad, nch)
    new_node, new_node_pad = _tc_add_partials(node_feat, parts, n_pad,
                                              n_pad // 16)
    bi2, bj2 = _sc_gather2(new_node_pad, idx2, e, nch)
    new_edge_t = _tc_edge_mlp(bi2, angt, auxt16, bj2, wge, we1t, ge1t,
                              bhge, be1r, ge1r, 3200)
    return new_node, new_edge_t.T


# speedup vs baseline: 4.0515x; 1.0597x over previous
import functools

import jax
import jax.numpy as jnp
from jax import lax
from jax.experimental import pallas as pl
from jax.experimental.pallas import tpu as pltpu
from jax.experimental.pallas import tpu_sc as plsc

NC = 2
NS = 16
NW = NC * NS
CH = 128
BF = jnp.bfloat16


def _pack16(x):
    lo = lax.bitcast_convert_type(x[:, :64].astype(BF), jnp.uint16)
    hi = lax.bitcast_convert_type(x[:, 64:].astype(BF), jnp.uint16)
    return (lo.astype(jnp.int32) | (hi.astype(jnp.int32) << 16))


def _unpack16(w):
    u = lax.bitcast_convert_type(w, jnp.uint32)
    lo = lax.bitcast_convert_type((u & jnp.uint32(0xFFFF)).astype(jnp.uint16), BF)
    hi = lax.bitcast_convert_type((u >> 16).astype(jnp.uint16), BF)
    return jnp.concatenate([lo, hi], axis=1)


def _sc_gather2(table_pad, idx2, e_rows, nch):
    n_pad, d = table_pad.shape
    nw, nt, ch = idx2.shape
    npt = n_pad // NS
    mesh = plsc.VectorSubcoreMesh(core_axis_name="c", subcore_axis_name="s")

    @functools.partial(
        pl.kernel,
        out_type=(
            jax.ShapeDtypeStruct((e_rows, d), jnp.float32),
            jax.ShapeDtypeStruct((e_rows, d), jnp.float32),
        ),
        mesh=mesh,
        scratch_types=[
            pltpu.VMEM((nt, ch), jnp.int32),
            pltpu.VMEM((2, ch, d), jnp.float32),
            pltpu.VMEM_SHARED((n_pad, d), jnp.float32),
            pltpu.SemaphoreType.DMA,
            pltpu.SemaphoreType.DMA,
        ],
    )
    def k(table_h, idx_h, oi_h, oj_h, idx_v, buf_v, tab_s, sem0, sem1):
        ci = lax.axis_index("c")
        si = lax.axis_index("s")
        wid = si * NC + ci
        pltpu.sync_copy(table_h.at[pl.ds(si * npt, npt)],
                        tab_s.at[pl.ds(si * npt, npt)])
        pltpu.sync_copy(idx_h.at[wid], idx_v)
        plsc.subcore_barrier()

        nt_w = 2 * ((nch - wid + NW - 1) // NW)

        def off(k_):
            return (wid + k_ * NW) * ch

        pltpu.async_copy(tab_s.at[idx_v.at[0]], buf_v.at[0], sem0)
        pltpu.async_copy(tab_s.at[idx_v.at[1]], buf_v.at[1], sem1)

        def body(t, carry):
            p = t % 2
            c = t // 2

            @pl.when(p == 0)
            def _even():
                pltpu.make_async_copy(tab_s.at[idx_v.at[t]], buf_v.at[0],
                                      sem0).wait()
                pltpu.sync_copy(buf_v.at[0], oi_h.at[pl.ds(off(c), ch)])
                @pl.when(t + 2 < nt_w)
                def _fire():
                    pltpu.async_copy(tab_s.at[idx_v.at[t + 2]], buf_v.at[0],
                                     sem0)

            @pl.when(p == 1)
            def _odd():
                pltpu.make_async_copy(tab_s.at[idx_v.at[t]], buf_v.at[1],
                                      sem1).wait()
                pltpu.sync_copy(buf_v.at[1], oj_h.at[pl.ds(off(c), ch)])
                @pl.when(t + 2 < nt_w)
                def _fire():
                    pltpu.async_copy(tab_s.at[idx_v.at[t + 2]], buf_v.at[1],
                                     sem1)

            return carry

        lax.fori_loop(0, nt_w, body, 0)

    return k(table_pad, idx2)


def _sc_scatter_add(msg, dst3, zeros, n_pad, nch):
    e_rows, d = msg.shape
    nw, kmax, ch = dst3.shape
    npt = n_pad // NS
    mesh = plsc.VectorSubcoreMesh(core_axis_name="c", subcore_axis_name="s")

    @functools.partial(
        pl.kernel,
        out_type=jax.ShapeDtypeStruct((NC, n_pad, d), jnp.float32),
        mesh=mesh,
        scratch_types=[
            pltpu.VMEM((kmax, ch), jnp.int32),
            pltpu.VMEM((2, ch, d), jnp.float32),
            pltpu.VMEM_SHARED((n_pad, d), jnp.float32),
            pltpu.SemaphoreType.DMA,
        ],
    )
    def k(msg_h, dst_h, zeros_h, out_h, idx_v, rows_v, acc_s, sem):
        ci = lax.axis_index("c")
        si = lax.axis_index("s")
        wid = si * NC + ci
        pltpu.sync_copy(zeros_h.at[pl.ds(si * npt, npt)],
                        acc_s.at[pl.ds(si * npt, npt)])
        plsc.subcore_barrier()
        pltpu.sync_copy(dst_h.at[wid], idx_v)

        nk_w = (nch - wid + NW - 1) // NW

        def off(k_):
            return (wid + k_ * NW) * ch

        pltpu.async_copy(msg_h.at[pl.ds(off(0), ch)], rows_v.at[0], sem)

        def body(c, carry):
            p = c % 2

            @pl.when(c + 1 < nk_w)
            def _fire_next():
                pltpu.async_copy(msg_h.at[pl.ds(off(c + 1), ch)],
                                 rows_v.at[1 - p], sem)

            pltpu.make_async_copy(msg_h.at[pl.ds(off(c), ch)], rows_v.at[p],
                                  sem).wait()
            pltpu.sync_copy(rows_v.at[p], acc_s.at[idx_v.at[c]], add=True)
            return carry

        lax.fori_loop(0, nk_w, body, 0)
        plsc.subcore_barrier()
        pltpu.sync_copy(acc_s.at[pl.ds(si * npt, npt)],
                        out_h.at[ci, pl.ds(si * npt, npt)])

    return k(msg, dst3, zeros)


def _tc_node_mlp(bi, angt, auxt, bj, wg0, w1t, g1t, woutt, bhg, b1r, g1r,
                 block_rows, blk0=0):
    e_rows, dn = bi.shape
    da = angt.shape[0]
    dx = auxt.shape[0]
    h = w1t.shape[0]
    grid = e_rows // block_rows

    def body(bi_r, angt_r, auxt_r, bj_r, wg0_r, w1_r, g1_r, wout_r, bhg_r,
             b1_r, g1b_r, o_r):
        xc = jnp.concatenate(
            [bi_r[...].astype(BF), angt_r[...].T, auxt_r[...].T,
             bj_r[...].astype(BF)], axis=1)
        x = jnp.dot(xc, wg0_r[...], preferred_element_type=jnp.float32)
        x += bhg_r[...]
        a = x * jax.nn.sigmoid(x)
        ah = a[:, :h].astype(BF)
        ag = a[:, h:].astype(BF)
        h2 = jnp.dot(ah, w1_r[...], preferred_element_type=jnp.float32) + b1_r[...]
        h2 = h2 * jax.nn.sigmoid(h2)
        g2 = jax.nn.sigmoid(
            jnp.dot(ag, g1_r[...], preferred_element_type=jnp.float32) + g1b_r[...])
        m = (h2 * g2).astype(BF)
        o_r[...] = jnp.dot(m, wout_r[...], preferred_element_type=jnp.float32)

    full = lambda i: (0, 0)
    return pl.pallas_call(
        body,
        grid=(grid,),
        in_specs=[
            pl.BlockSpec((block_rows, dn), lambda i: (i, 0)),
            pl.BlockSpec((da, block_rows), lambda i: (0, i + blk0)),
            pl.BlockSpec((dx, block_rows), lambda i: (0, i + blk0)),
            pl.BlockSpec((block_rows, dn), lambda i: (i, 0)),
            pl.BlockSpec(wg0.shape, full),
            pl.BlockSpec(w1t.shape, full),
            pl.BlockSpec(g1t.shape, full),
            pl.BlockSpec(woutt.shape, full),
            pl.BlockSpec(bhg.shape, full),
            pl.BlockSpec(b1r.shape, full),
            pl.BlockSpec(g1r.shape, full),
        ],
        out_specs=pl.BlockSpec((block_rows, dn), lambda i: (i, 0)),
        out_shape=jax.ShapeDtypeStruct((e_rows, dn), jnp.float32),
    )(bi, angt, auxt, bj, wg0, w1t, g1t, woutt, bhg, b1r, g1r)


def _tc_edge_mlp(bi, angt, auxt, bj, wge, w1t, g1t, bhg, b1r, g1r,
                 block_rows):
    e_rows, dn = bi.shape
    da = angt.shape[0]
    dx = auxt.shape[0]
    h = w1t.shape[0]
    grid = e_rows // block_rows

    def body(bi_r, angt_r, auxt_r, bj_r, wge_r, w1_r, g1_r, bhg_r, b1_r,
             g1b_r, o_r):
        ang = angt_r[...].T
        xc = jnp.concatenate(
            [bi_r[...].astype(BF), ang.astype(BF), auxt_r[...].T,
             bj_r[...].astype(BF)], axis=1)
        x = jnp.dot(xc, wge_r[...], preferred_element_type=jnp.float32)
        x += bhg_r[...]
        a = x * jax.nn.sigmoid(x)
        ah = a[:, :h].astype(BF)
        ag = a[:, h:].astype(BF)
        h2 = jnp.dot(ah, w1_r[...], preferred_element_type=jnp.float32) + b1_r[...]
        h2 = h2 * jax.nn.sigmoid(h2)
        g2 = jax.nn.sigmoid(
            jnp.dot(ag, g1_r[...], preferred_element_type=jnp.float32) + g1b_r[...])
        o_r[...] = (ang + h2 * g2).T

    full = lambda i: (0, 0)
    return pl.pallas_call(
        body,
        grid=(grid,),
        in_specs=[
            pl.BlockSpec((block_rows, dn), lambda i: (i, 0)),
            pl.BlockSpec((da, block_rows), lambda i: (0, i)),
            pl.BlockSpec((dx, block_rows), lambda i: (0, i)),
            pl.BlockSpec((block_rows, dn), lambda i: (i, 0)),
            pl.BlockSpec(wge.shape, full),
            pl.BlockSpec(w1t.shape, full),
            pl.BlockSpec(g1t.shape, full),
            pl.BlockSpec(bhg.shape, full),
            pl.BlockSpec(b1r.shape, full),
            pl.BlockSpec(g1r.shape, full),
        ],
        out_specs=pl.BlockSpec((da, block_rows), lambda i: (0, i)),
        out_shape=jax.ShapeDtypeStruct((da, e_rows), jnp.float32),
    )(bi, angt, auxt, bj, wge, w1t, g1t, bhg, b1r, g1r)


def _tc_add_partials(node_feat, parts_a, parts_b, n_pad, block_rows):
    n, d = node_feat.shape
    grid = n_pad // block_rows

    def body(nf_r, pa_r, pb_r, o_r, op_r):
        o = nf_r[...] + (pa_r[0] + pa_r[1]) + (pb_r[0] + pb_r[1])
        o_r[...] = o
        op_r[...] = o

    return pl.pallas_call(
        body,
        grid=(grid,),
        in_specs=[
            pl.BlockSpec((block_rows, d), lambda i: (i, 0)),
            pl.BlockSpec((NC, block_rows, d), lambda i: (0, i, 0)),
            pl.BlockSpec((NC, block_rows, d), lambda i: (0, i, 0)),
        ],
        out_specs=[
            pl.BlockSpec((block_rows, d), lambda i: (i, 0)),
            pl.BlockSpec((block_rows, d), lambda i: (i, 0)),
        ],
        out_shape=[
            jax.ShapeDtypeStruct((n, d), jnp.float32),
            jax.ShapeDtypeStruct((n_pad, d), jnp.float32),
        ],
    )(node_feat, parts_a, parts_b)


def kernel(node_feat, angle_feat, aux_feat, edge_index, W0, b0, W1, b1, G0,
           g0, G1, g1, Wout, We0, be0, We1, be1, Ge0, ge0, Ge1, ge1):
    n, dn = node_feat.shape
    e, da = angle_feat.shape
    dx = aux_feat.shape[1]
    h = W0.shape[0]

    nch = e // CH
    eh = e // 2
    nch_h = eh // CH
    kmax = -(-nch // NW)
    kmax_h = -(-nch_h // NW)
    n_grain = 8 * NS
    n_pad = n_grain * (-(-n // n_grain))

    src = edge_index[0]
    dst = edge_index[1]

    def chunks3(v, km):
        nc_ = v.shape[0] // CH
        vp = jnp.pad(v, (0, km * NW * CH - v.shape[0]))
        return vp.reshape(km, NW, CH).transpose(1, 0, 2)

    def idx2_of(s_, d_, km):
        s3 = chunks3(s_, km)
        d3 = chunks3(d_, km)
        return jnp.stack([s3, d3], axis=2).reshape(NW, 2 * km, CH), d3

    idx2a, dst3a = idx2_of(src[:eh], dst[:eh], kmax_h)
    idx2b, dst3b = idx2_of(src[eh:], dst[eh:], kmax_h)
    idx2, _ = idx2_of(src, dst, kmax)

    node_pad = jnp.pad(node_feat, ((0, n_pad - n), (0, 0)))
    zeros_n = jnp.zeros((n_pad, dn), jnp.float32)
    angt = angle_feat.T
    auxt = aux_feat.T
    angt16 = angt.astype(BF)
    auxt16 = auxt.astype(BF)

    wg0 = jnp.concatenate([W0.T, G0.T], axis=1).astype(BF)
    bhg0 = jnp.concatenate([b0, g0]).reshape(1, 2 * h)
    w1t = W1.T.astype(BF)
    g1t = G1.T.astype(BF)
    woutt = Wout.T.astype(BF)
    b1r = b1.reshape(1, -1)
    g1r = g1.reshape(1, -1)

    wge = jnp.concatenate([We0.T, Ge0.T], axis=1).astype(BF)
    bhge = jnp.concatenate([be0, ge0]).reshape(1, 2 * h)
    we1t = We1.T.astype(BF)
    ge1t = Ge1.T.astype(BF)
    be1r = be1.reshape(1, -1)
    ge1r = ge1.reshape(1, -1)

    bia, bja = _sc_gather2(node_pad, idx2a, eh, nch_h)
    bib, bjb = _sc_gather2(node_pad, idx2b, eh, nch_h)
    msgwa = _tc_node_mlp(bia, angt16, auxt16, bja, wg0, w1t,
                         g1t, woutt, bhg0, b1r, g1r, 3200, 0)
    partsa = _sc_scatter_add(msgwa, dst3a, zeros_n, n_pad, nch_h)
    msgwb = _tc_node_mlp(bib, angt16, auxt16, bjb, wg0, w1t,
                         g1t, woutt, bhg0, b1r, g1r, 3200, eh // 3200)
    partsb = _sc_scatter_add(msgwb, dst3b, zeros_n, n_pad, nch_h)
    new_node, new_node_pad = _tc_add_partials(node_feat, partsa, partsb,
                                              n_pad, n_pad // 16)
    bi2, bj2 = _sc_gather2(new_node_pad, idx2, e, nch)
    new_edge_t = _tc_edge_mlp(bi2, angt, auxt16, bj2, wge, we1t, ge1t,
                              bhge, be1r, ge1r, 3200)
    return new_node, new_edge_t.T


# speedup vs baseline: 4.0985x; 1.0116x over previous
import functools

import jax
import jax.numpy as jnp
from jax import lax
from jax.experimental import pallas as pl
from jax.experimental.pallas import tpu as pltpu
from jax.experimental.pallas import tpu_sc as plsc

NC = 2
NS = 16
NW = NC * NS
CH = 128
BF = jnp.bfloat16


def _pack16(x):
    lo = lax.bitcast_convert_type(x[:, :64].astype(BF), jnp.uint16)
    hi = lax.bitcast_convert_type(x[:, 64:].astype(BF), jnp.uint16)
    return (lo.astype(jnp.int32) | (hi.astype(jnp.int32) << 16))


def _unpack16(w):
    u = lax.bitcast_convert_type(w, jnp.uint32)
    lo = lax.bitcast_convert_type((u & jnp.uint32(0xFFFF)).astype(jnp.uint16), BF)
    hi = lax.bitcast_convert_type((u >> 16).astype(jnp.uint16), BF)
    return jnp.concatenate([lo, hi], axis=1)


def _sc_gather2(table_pad, idx2, e_rows, nch):
    n_pad, d = table_pad.shape
    nw, nt, ch = idx2.shape
    npt = n_pad // NS
    mesh = plsc.VectorSubcoreMesh(core_axis_name="c", subcore_axis_name="s")

    @functools.partial(
        pl.kernel,
        out_type=(
            jax.ShapeDtypeStruct((e_rows, d), jnp.float32),
            jax.ShapeDtypeStruct((e_rows, d), jnp.float32),
        ),
        mesh=mesh,
        scratch_types=[
            pltpu.VMEM((nt, ch), jnp.int32),
            pltpu.VMEM((2, ch, d), jnp.float32),
            pltpu.VMEM_SHARED((n_pad, d), jnp.float32),
            pltpu.SemaphoreType.DMA,
            pltpu.SemaphoreType.DMA,
        ],
    )
    def k(table_h, idx_h, oi_h, oj_h, idx_v, buf_v, tab_s, sem0, sem1):
        ci = lax.axis_index("c")
        si = lax.axis_index("s")
        wid = si * NC + ci
        pltpu.sync_copy(table_h.at[pl.ds(si * npt, npt)],
                        tab_s.at[pl.ds(si * npt, npt)])
        pltpu.sync_copy(idx_h.at[wid], idx_v)
        plsc.subcore_barrier()

        nt_w = 2 * ((nch - wid + NW - 1) // NW)

        def off(k_):
            return (wid + k_ * NW) * ch

        pltpu.async_copy(tab_s.at[idx_v.at[0]], buf_v.at[0], sem0)
        pltpu.async_copy(tab_s.at[idx_v.at[1]], buf_v.at[1], sem1)

        def body(t, carry):
            p = t % 2
            c = t // 2

            @pl.when(p == 0)
            def _even():
                pltpu.make_async_copy(tab_s.at[idx_v.at[t]], buf_v.at[0],
                                      sem0).wait()
                pltpu.sync_copy(buf_v.at[0], oi_h.at[pl.ds(off(c), ch)])
                @pl.when(t + 2 < nt_w)
                def _fire():
                    pltpu.async_copy(tab_s.at[idx_v.at[t + 2]], buf_v.at[0],
                                     sem0)

            @pl.when(p == 1)
            def _odd():
                pltpu.make_async_copy(tab_s.at[idx_v.at[t]], buf_v.at[1],
                                      sem1).wait()
                pltpu.sync_copy(buf_v.at[1], oj_h.at[pl.ds(off(c), ch)])
                @pl.when(t + 2 < nt_w)
                def _fire():
                    pltpu.async_copy(tab_s.at[idx_v.at[t + 2]], buf_v.at[1],
                                     sem1)

            return carry

        lax.fori_loop(0, nt_w, body, 0)

    return k(table_pad, idx2)


def _sc_scatter_add(msg, dst3, zeros, n_pad, nch):
    e_rows, d = msg.shape
    nw, kmax, ch = dst3.shape
    npt = n_pad // NS
    mesh = plsc.VectorSubcoreMesh(core_axis_name="c", subcore_axis_name="s")

    @functools.partial(
        pl.kernel,
        out_type=jax.ShapeDtypeStruct((NC, n_pad, d), jnp.float32),
        mesh=mesh,
        scratch_types=[
            pltpu.VMEM((kmax, ch), jnp.int32),
            pltpu.VMEM((2, ch, d), jnp.float32),
            pltpu.VMEM_SHARED((n_pad, d), jnp.float32),
            pltpu.SemaphoreType.DMA,
        ],
    )
    def k(msg_h, dst_h, zeros_h, out_h, idx_v, rows_v, acc_s, sem):
        ci = lax.axis_index("c")
        si = lax.axis_index("s")
        wid = si * NC + ci
        pltpu.sync_copy(zeros_h.at[pl.ds(si * npt, npt)],
                        acc_s.at[pl.ds(si * npt, npt)])
        plsc.subcore_barrier()
        pltpu.sync_copy(dst_h.at[wid], idx_v)

        nk_w = (nch - wid + NW - 1) // NW

        def off(k_):
            return (wid + k_ * NW) * ch

        pltpu.async_copy(msg_h.at[pl.ds(off(0), ch)], rows_v.at[0], sem)

        def body(c, carry):
            p = c % 2

            @pl.when(c + 1 < nk_w)
            def _fire_next():
                pltpu.async_copy(msg_h.at[pl.ds(off(c + 1), ch)],
                                 rows_v.at[1 - p], sem)

            pltpu.make_async_copy(msg_h.at[pl.ds(off(c), ch)], rows_v.at[p],
                                  sem).wait()
            pltpu.sync_copy(rows_v.at[p], acc_s.at[idx_v.at[c]], add=True)
            return carry

        lax.fori_loop(0, nk_w, body, 0)
        plsc.subcore_barrier()
        pltpu.sync_copy(acc_s.at[pl.ds(si * npt, npt)],
                        out_h.at[ci, pl.ds(si * npt, npt)])

    return k(msg, dst3, zeros)


def _tc_node_mlp(bi, angt, auxt, bj, wg0, w1t, g1t, woutt, bhg, b1r, g1r,
                 block_rows, blk0=0):
    e_rows, dn = bi.shape
    da = angt.shape[0]
    dx = auxt.shape[0]
    h = w1t.shape[0]
    grid = e_rows // block_rows

    def body(bi_r, angt_r, auxt_r, bj_r, wg0_r, w1_r, g1_r, wout_r, bhg_r,
             b1_r, g1b_r, o_r):
        xc = jnp.concatenate(
            [bi_r[...].astype(BF), angt_r[...].T, auxt_r[...].T,
             bj_r[...].astype(BF)], axis=1)
        x = jnp.dot(xc, wg0_r[...], preferred_element_type=jnp.float32)
        x += bhg_r[...]
        a = x * jax.nn.sigmoid(x)
        ah = a[:, :h].astype(BF)
        ag = a[:, h:].astype(BF)
        h2 = jnp.dot(ah, w1_r[...], preferred_element_type=jnp.float32) + b1_r[...]
        h2 = h2 * jax.nn.sigmoid(h2)
        g2 = jax.nn.sigmoid(
            jnp.dot(ag, g1_r[...], preferred_element_type=jnp.float32) + g1b_r[...])
        m = (h2 * g2).astype(BF)
        o_r[...] = jnp.dot(m, wout_r[...], preferred_element_type=jnp.float32)

    full = lambda i: (0, 0)
    return pl.pallas_call(
        body,
        grid=(grid,),
        in_specs=[
            pl.BlockSpec((block_rows, dn), lambda i: (i, 0)),
            pl.BlockSpec((da, block_rows), lambda i: (0, i + blk0)),
            pl.BlockSpec((dx, block_rows), lambda i: (0, i + blk0)),
            pl.BlockSpec((block_rows, dn), lambda i: (i, 0)),
            pl.BlockSpec(wg0.shape, full),
            pl.BlockSpec(w1t.shape, full),
            pl.BlockSpec(g1t.shape, full),
            pl.BlockSpec(woutt.shape, full),
            pl.BlockSpec(bhg.shape, full),
            pl.BlockSpec(b1r.shape, full),
            pl.BlockSpec(g1r.shape, full),
        ],
        out_specs=pl.BlockSpec((block_rows, dn), lambda i: (i, 0)),
        out_shape=jax.ShapeDtypeStruct((e_rows, dn), jnp.float32),
    )(bi, angt, auxt, bj, wg0, w1t, g1t, woutt, bhg, b1r, g1r)


def _tc_edge_mlp(bi, angt, auxt, bj, wge, w1t, g1t, bhg, b1r, g1r,
                 block_rows):
    e_rows, dn = bi.shape
    da = angt.shape[0]
    dx = auxt.shape[0]
    h = w1t.shape[0]
    grid = e_rows // block_rows

    def body(bi_r, angt_r, auxt_r, bj_r, wge_r, w1_r, g1_r, bhg_r, b1_r,
             g1b_r, o_r):
        ang = angt_r[...].T
        xc = jnp.concatenate(
            [bi_r[...].astype(BF), ang.astype(BF), auxt_r[...].T,
             bj_r[...].astype(BF)], axis=1)
        x = jnp.dot(xc, wge_r[...], preferred_element_type=jnp.float32)
        x += bhg_r[...]
        a = x * jax.nn.sigmoid(x)
        ah = a[:, :h].astype(BF)
        ag = a[:, h:].astype(BF)
        h2 = jnp.dot(ah, w1_r[...], preferred_element_type=jnp.float32) + b1_r[...]
        h2 = h2 * jax.nn.sigmoid(h2)
        g2 = jax.nn.sigmoid(
            jnp.dot(ag, g1_r[...], preferred_element_type=jnp.float32) + g1b_r[...])
        o_r[...] = (ang + h2 * g2).T

    full = lambda i: (0, 0)
    return pl.pallas_call(
        body,
        grid=(grid,),
        in_specs=[
            pl.BlockSpec((block_rows, dn), lambda i: (i, 0)),
            pl.BlockSpec((da, block_rows), lambda i: (0, i)),
            pl.BlockSpec((dx, block_rows), lambda i: (0, i)),
            pl.BlockSpec((block_rows, dn), lambda i: (i, 0)),
            pl.BlockSpec(wge.shape, full),
            pl.BlockSpec(w1t.shape, full),
            pl.BlockSpec(g1t.shape, full),
            pl.BlockSpec(bhg.shape, full),
            pl.BlockSpec(b1r.shape, full),
            pl.BlockSpec(g1r.shape, full),
        ],
        out_specs=pl.BlockSpec((da, block_rows), lambda i: (0, i)),
        out_shape=jax.ShapeDtypeStruct((da, e_rows), jnp.float32),
    )(bi, angt, auxt, bj, wge, w1t, g1t, bhg, b1r, g1r)


def _tc_add_partials(node_feat, parts_a, parts_b, n_pad, block_rows):
    n, d = node_feat.shape
    grid = n_pad // block_rows

    def body(nf_r, pa_r, pb_r, o_r, op_r):
        o = nf_r[...] + (pa_r[0] + pa_r[1]) + (pb_r[0] + pb_r[1])
        o_r[...] = o
        op_r[...] = o

    return pl.pallas_call(
        body,
        grid=(grid,),
        in_specs=[
            pl.BlockSpec((block_rows, d), lambda i: (i, 0)),
            pl.BlockSpec((NC, block_rows, d), lambda i: (0, i, 0)),
            pl.BlockSpec((NC, block_rows, d), lambda i: (0, i, 0)),
        ],
        out_specs=[
            pl.BlockSpec((block_rows, d), lambda i: (i, 0)),
            pl.BlockSpec((block_rows, d), lambda i: (i, 0)),
        ],
        out_shape=[
            jax.ShapeDtypeStruct((n, d), jnp.float32),
            jax.ShapeDtypeStruct((n_pad, d), jnp.float32),
        ],
    )(node_feat, parts_a, parts_b)


def kernel(node_feat, angle_feat, aux_feat, edge_index, W0, b0, W1, b1, G0,
           g0, G1, g1, Wout, We0, be0, We1, be1, Ge0, ge0, Ge1, ge1):
    n, dn = node_feat.shape
    e, da = angle_feat.shape
    dx = aux_feat.shape[1]
    h = W0.shape[0]

    nch = e // CH
    eh = e // 2
    nch_h = eh // CH
    kmax = -(-nch // NW)
    kmax_h = -(-nch_h // NW)
    n_grain = 8 * NS
    n_pad = n_grain * (-(-n // n_grain))

    src = edge_index[0]
    dst = edge_index[1]

    def chunks3(v, km):
        nc_ = v.shape[0] // CH
        vp = jnp.pad(v, (0, km * NW * CH - v.shape[0]))
        return vp.reshape(km, NW, CH).transpose(1, 0, 2)

    def idx2_of(s_, d_, km):
        s3 = chunks3(s_, km)
        d3 = chunks3(d_, km)
        return jnp.stack([s3, d3], axis=2).reshape(NW, 2 * km, CH), d3

    idx2a, dst3a = idx2_of(src[:eh], dst[:eh], kmax_h)
    idx2b, dst3b = idx2_of(src[eh:], dst[eh:], kmax_h)
    idx2, _ = idx2_of(src, dst, kmax)

    node_pad = jnp.pad(node_feat, ((0, n_pad - n), (0, 0)))
    zeros_n = jnp.zeros((n_pad, dn), jnp.float32)
    angt = angle_feat.T
    auxt = aux_feat.T
    angt16 = angt.astype(BF)
    auxt16 = auxt.astype(BF)

    wg0 = jnp.concatenate([W0.T, G0.T], axis=1).astype(BF)
    bhg0 = jnp.concatenate([b0, g0]).reshape(1, 2 * h)
    w1t = W1.T.astype(BF)
    g1t = G1.T.astype(BF)
    woutt = Wout.T.astype(BF)
    b1r = b1.reshape(1, -1)
    g1r = g1.reshape(1, -1)

    wge = jnp.concatenate([We0.T, Ge0.T], axis=1).astype(BF)
    bhge = jnp.concatenate([be0, ge0]).reshape(1, 2 * h)
    we1t = We1.T.astype(BF)
    ge1t = Ge1.T.astype(BF)
    be1r = be1.reshape(1, -1)
    ge1r = ge1.reshape(1, -1)

    bia, bja = _sc_gather2(node_pad, idx2a, eh, nch_h)
    bib, bjb = _sc_gather2(node_pad, idx2b, eh, nch_h)
    msgwa = _tc_node_mlp(bia, angt16, auxt16, bja, wg0, w1t,
                         g1t, woutt, bhg0, b1r, g1r, 3200, 0)
    partsa = _sc_scatter_add(msgwa, dst3a, zeros_n, n_pad, nch_h)
    msgwb = _tc_node_mlp(bib, angt16, auxt16, bjb, wg0, w1t,
                         g1t, woutt, bhg0, b1r, g1r, 3200, eh // 3200)
    partsb = _sc_scatter_add(msgwb, dst3b, zeros_n, n_pad, nch_h)
    new_node, new_node_pad = _tc_add_partials(node_feat, partsa, partsb,
                                              n_pad, n_pad // 16)
    bi2, bj2 = _sc_gather2(new_node_pad, idx2, e, nch)
    new_edge_t = _tc_edge_mlp(bi2, angt, auxt16, bj2, wge, we1t, ge1t,
                              bhge, be1r, ge1r, 6400)
    return new_node, new_edge_t.T


# speedup vs baseline: 4.1108x; 1.0030x over previous
import functools

import jax
import jax.numpy as jnp
from jax import lax
from jax.experimental import pallas as pl
from jax.experimental.pallas import tpu as pltpu
from jax.experimental.pallas import tpu_sc as plsc

NC = 2
NS = 16
NW = NC * NS
CH = 128
BF = jnp.bfloat16


def _pack16(x):
    lo = lax.bitcast_convert_type(x[:, :64].astype(BF), jnp.uint16)
    hi = lax.bitcast_convert_type(x[:, 64:].astype(BF), jnp.uint16)
    return (lo.astype(jnp.int32) | (hi.astype(jnp.int32) << 16))


def _unpack16(w):
    u = lax.bitcast_convert_type(w, jnp.uint32)
    lo = lax.bitcast_convert_type((u & jnp.uint32(0xFFFF)).astype(jnp.uint16), BF)
    hi = lax.bitcast_convert_type((u >> 16).astype(jnp.uint16), BF)
    return jnp.concatenate([lo, hi], axis=1)


def _sc_gather2(table_pad, idx2, e_rows, nch):
    n_pad, d = table_pad.shape
    nw, nt, ch = idx2.shape
    npt = n_pad // NS
    mesh = plsc.VectorSubcoreMesh(core_axis_name="c", subcore_axis_name="s")

    @functools.partial(
        pl.kernel,
        out_type=(
            jax.ShapeDtypeStruct((e_rows, d), jnp.float32),
            jax.ShapeDtypeStruct((e_rows, d), jnp.float32),
        ),
        mesh=mesh,
        scratch_types=[
            pltpu.VMEM((nt, ch), jnp.int32),
            pltpu.VMEM((2, ch, d), jnp.float32),
            pltpu.VMEM_SHARED((n_pad, d), jnp.float32),
            pltpu.SemaphoreType.DMA,
            pltpu.SemaphoreType.DMA,
        ],
    )
    def k(table_h, idx_h, oi_h, oj_h, idx_v, buf_v, tab_s, sem0, sem1):
        ci = lax.axis_index("c")
        si = lax.axis_index("s")
        wid = si * NC + ci
        pltpu.sync_copy(table_h.at[pl.ds(si * npt, npt)],
                        tab_s.at[pl.ds(si * npt, npt)])
        pltpu.sync_copy(idx_h.at[wid], idx_v)
        plsc.subcore_barrier()

        nt_w = 2 * ((nch - wid + NW - 1) // NW)

        def off(k_):
            return (wid + k_ * NW) * ch

        pltpu.async_copy(tab_s.at[idx_v.at[0]], buf_v.at[0], sem0)
        pltpu.async_copy(tab_s.at[idx_v.at[1]], buf_v.at[1], sem1)

        def body(t, carry):
            p = t % 2
            c = t // 2

            @pl.when(p == 0)
            def _even():
                pltpu.make_async_copy(tab_s.at[idx_v.at[t]], buf_v.at[0],
                                      sem0).wait()
                pltpu.sync_copy(buf_v.at[0], oi_h.at[pl.ds(off(c), ch)])
                @pl.when(t + 2 < nt_w)
                def _fire():
                    pltpu.async_copy(tab_s.at[idx_v.at[t + 2]], buf_v.at[0],
                                     sem0)

            @pl.when(p == 1)
            def _odd():
                pltpu.make_async_copy(tab_s.at[idx_v.at[t]], buf_v.at[1],
                                      sem1).wait()
                pltpu.sync_copy(buf_v.at[1], oj_h.at[pl.ds(off(c), ch)])
                @pl.when(t + 2 < nt_w)
                def _fire():
                    pltpu.async_copy(tab_s.at[idx_v.at[t + 2]], buf_v.at[1],
                                     sem1)

            return carry

        lax.fori_loop(0, nt_w, body, 0)

    return k(table_pad, idx2)


def _sc_scatter_add(msg, dst3, zeros, n_pad, nch):
    e_rows, d = msg.shape
    nw, kmax, ch = dst3.shape
    npt = n_pad // NS
    mesh = plsc.VectorSubcoreMesh(core_axis_name="c", subcore_axis_name="s")

    @functools.partial(
        pl.kernel,
        out_type=jax.ShapeDtypeStruct((NC, n_pad, d), jnp.float32),
        mesh=mesh,
        scratch_types=[
            pltpu.VMEM((kmax, ch), jnp.int32),
            pltpu.VMEM((2, ch, d), jnp.float32),
            pltpu.VMEM_SHARED((n_pad, d), jnp.float32),
            pltpu.SemaphoreType.DMA,
        ],
    )
    def k(msg_h, dst_h, zeros_h, out_h, idx_v, rows_v, acc_s, sem):
        ci = lax.axis_index("c")
        si = lax.axis_index("s")
        wid = si * NC + ci
        pltpu.sync_copy(zeros_h.at[pl.ds(si * npt, npt)],
                        acc_s.at[pl.ds(si * npt, npt)])
        plsc.subcore_barrier()
        pltpu.sync_copy(dst_h.at[wid], idx_v)

        nk_w = (nch - wid + NW - 1) // NW

        def off(k_):
            return (wid + k_ * NW) * ch

        pltpu.async_copy(msg_h.at[pl.ds(off(0), ch)], rows_v.at[0], sem)

        def body(c, carry):
            p = c % 2

            @pl.when(c + 1 < nk_w)
            def _fire_next():
                pltpu.async_copy(msg_h.at[pl.ds(off(c + 1), ch)],
                                 rows_v.at[1 - p], sem)

            pltpu.make_async_copy(msg_h.at[pl.ds(off(c), ch)], rows_v.at[p],
                                  sem).wait()
            pltpu.sync_copy(rows_v.at[p], acc_s.at[idx_v.at[c]], add=True)
            return carry

        lax.fori_loop(0, nk_w, body, 0)
        plsc.subcore_barrier()
        pltpu.sync_copy(acc_s.at[pl.ds(si * npt, npt)],
                        out_h.at[ci, pl.ds(si * npt, npt)])

    return k(msg, dst3, zeros)


def _tc_node_mlp(bi, angt, auxt, bj, wg0, w1t, g1t, woutt, bhg, b1r, g1r,
                 block_rows, blk0=0):
    e_rows, dn = bi.shape
    da = angt.shape[0]
    dx = auxt.shape[0]
    h = w1t.shape[0]
    grid = e_rows // block_rows

    def body(bi_r, angt_r, auxt_r, bj_r, wg0_r, w1_r, g1_r, wout_r, bhg_r,
             b1_r, g1b_r, o_r):
        xc = jnp.concatenate(
            [bi_r[...].astype(BF), angt_r[...].T, auxt_r[...].T,
             bj_r[...].astype(BF)], axis=1)
        x = jnp.dot(xc, wg0_r[...], preferred_element_type=jnp.float32)
        x += bhg_r[...]
        a = x * jax.nn.sigmoid(x)
        ah = a[:, :h].astype(BF)
        ag = a[:, h:].astype(BF)
        h2 = jnp.dot(ah, w1_r[...], preferred_element_type=jnp.float32) + b1_r[...]
        h2 = h2 * jax.nn.sigmoid(h2)
        g2 = jax.nn.sigmoid(
            jnp.dot(ag, g1_r[...], preferred_element_type=jnp.float32) + g1b_r[...])
        m = (h2 * g2).astype(BF)
        o_r[...] = jnp.dot(m, wout_r[...], preferred_element_type=jnp.float32)

    full = lambda i: (0, 0)
    return pl.pallas_call(
        body,
        grid=(grid,),
        in_specs=[
            pl.BlockSpec((block_rows, dn), lambda i: (i, 0)),
            pl.BlockSpec((da, block_rows), lambda i: (0, i + blk0)),
            pl.BlockSpec((dx, block_rows), lambda i: (0, i + blk0)),
            pl.BlockSpec((block_rows, dn), lambda i: (i, 0)),
            pl.BlockSpec(wg0.shape, full),
            pl.BlockSpec(w1t.shape, full),
            pl.BlockSpec(g1t.shape, full),
            pl.BlockSpec(woutt.shape, full),
            pl.BlockSpec(bhg.shape, full),
            pl.BlockSpec(b1r.shape, full),
            pl.BlockSpec(g1r.shape, full),
        ],
        out_specs=pl.BlockSpec((block_rows, dn), lambda i: (i, 0)),
        out_shape=jax.ShapeDtypeStruct((e_rows, dn), jnp.float32),
    )(bi, angt, auxt, bj, wg0, w1t, g1t, woutt, bhg, b1r, g1r)


def _tc_edge_mlp(bi, angt, auxt, bj, wge, w1t, g1t, bhg, b1r, g1r,
                 block_rows):
    e_rows, dn = bi.shape
    da = angt.shape[0]
    dx = auxt.shape[0]
    h = w1t.shape[0]
    grid = e_rows // block_rows

    def body(bi_r, angt_r, auxt_r, bj_r, wge_r, w1_r, g1_r, bhg_r, b1_r,
             g1b_r, o_r):
        ang = angt_r[...].T
        xc = jnp.concatenate(
            [bi_r[...].astype(BF), ang.astype(BF), auxt_r[...].T,
             bj_r[...].astype(BF)], axis=1)
        x = jnp.dot(xc, wge_r[...], preferred_element_type=jnp.float32)
        x += bhg_r[...]
        a = x * jax.nn.sigmoid(x)
        ah = a[:, :h].astype(BF)
        ag = a[:, h:].astype(BF)
        h2 = jnp.dot(ah, w1_r[...], preferred_element_type=jnp.float32) + b1_r[...]
        h2 = h2 * jax.nn.sigmoid(h2)
        g2 = jax.nn.sigmoid(
            jnp.dot(ag, g1_r[...], preferred_element_type=jnp.float32) + g1b_r[...])
        o_r[...] = (ang + h2 * g2).T

    full = lambda i: (0, 0)
    return pl.pallas_call(
        body,
        grid=(grid,),
        in_specs=[
            pl.BlockSpec((block_rows, dn), lambda i: (i, 0)),
            pl.BlockSpec((da, block_rows), lambda i: (0, i)),
            pl.BlockSpec((dx, block_rows), lambda i: (0, i)),
            pl.BlockSpec((block_rows, dn), lambda i: (i, 0)),
            pl.BlockSpec(wge.shape, full),
            pl.BlockSpec(w1t.shape, full),
            pl.BlockSpec(g1t.shape, full),
            pl.BlockSpec(bhg.shape, full),
            pl.BlockSpec(b1r.shape, full),
            pl.BlockSpec(g1r.shape, full),
        ],
        out_specs=pl.BlockSpec((da, block_rows), lambda i: (0, i)),
        out_shape=jax.ShapeDtypeStruct((da, e_rows), jnp.float32),
    )(bi, angt, auxt, bj, wge, w1t, g1t, bhg, b1r, g1r)


def _tc_add_partials(node_feat, parts_list, n_pad, block_rows):
    n, d = node_feat.shape
    grid = n_pad // block_rows
    np_ = len(parts_list)

    def body(nf_r, *refs):
        p_refs = refs[:np_]
        o_r, op_r = refs[np_], refs[np_ + 1]
        o = nf_r[...]
        for p_r in p_refs:
            o = o + (p_r[0] + p_r[1])
        o_r[...] = o
        op_r[...] = o

    return pl.pallas_call(
        body,
        grid=(grid,),
        in_specs=[
            pl.BlockSpec((block_rows, d), lambda i: (i, 0)),
        ] + [
            pl.BlockSpec((NC, block_rows, d), lambda i: (0, i, 0))
            for _ in range(np_)
        ],
        out_specs=[
            pl.BlockSpec((block_rows, d), lambda i: (i, 0)),
            pl.BlockSpec((block_rows, d), lambda i: (i, 0)),
        ],
        out_shape=[
            jax.ShapeDtypeStruct((n, d), jnp.float32),
            jax.ShapeDtypeStruct((n_pad, d), jnp.float32),
        ],
    )(node_feat, *parts_list)


def kernel(node_feat, angle_feat, aux_feat, edge_index, W0, b0, W1, b1, G0,
           g0, G1, g1, Wout, We0, be0, We1, be1, Ge0, ge0, Ge1, ge1):
    n, dn = node_feat.shape
    e, da = angle_feat.shape
    dx = aux_feat.shape[1]
    h = W0.shape[0]

    nch = e // CH
    parts_ch = [nch // 5, 2 * nch // 5, 2 * nch // 5]
    kmax = -(-nch // NW)
    n_grain = 8 * NS
    n_pad = n_grain * (-(-n // n_grain))

    src = edge_index[0]
    dst = edge_index[1]

    def chunks3(v, km):
        nc_ = v.shape[0] // CH
        vp = jnp.pad(v, (0, km * NW * CH - v.shape[0]))
        return vp.reshape(km, NW, CH).transpose(1, 0, 2)

    def idx2_of(s_, d_, km):
        s3 = chunks3(s_, km)
        d3 = chunks3(d_, km)
        return jnp.stack([s3, d3], axis=2).reshape(NW, 2 * km, CH), d3

    part_idx = []
    c0 = 0
    for pc in parts_ch:
        e0, e1 = c0 * CH, (c0 + pc) * CH
        km = -(-pc // NW)
        part_idx.append((e0, e1, pc) + idx2_of(src[e0:e1], dst[e0:e1], km))
        c0 += pc
    idx2, _ = idx2_of(src, dst, kmax)

    node_pad = jnp.pad(node_feat, ((0, n_pad - n), (0, 0)))
    zeros_n = jnp.zeros((n_pad, dn), jnp.float32)
    angt = angle_feat.T
    auxt = aux_feat.T
    angt16 = angt.astype(BF)
    auxt16 = auxt.astype(BF)

    wg0 = jnp.concatenate([W0.T, G0.T], axis=1).astype(BF)
    bhg0 = jnp.concatenate([b0, g0]).reshape(1, 2 * h)
    w1t = W1.T.astype(BF)
    g1t = G1.T.astype(BF)
    woutt = Wout.T.astype(BF)
    b1r = b1.reshape(1, -1)
    g1r = g1.reshape(1, -1)

    wge = jnp.concatenate([We0.T, Ge0.T], axis=1).astype(BF)
    bhge = jnp.concatenate([be0, ge0]).reshape(1, 2 * h)
    we1t = We1.T.astype(BF)
    ge1t = Ge1.T.astype(BF)
    be1r = be1.reshape(1, -1)
    ge1r = ge1.reshape(1, -1)

    NBLK = 6400
    bonds = [_sc_gather2(node_pad, p[3], p[1] - p[0], p[2])
             for p in part_idx]
    parts_acc = []
    for (e0, e1, pc, _i2, d3), (bi_, bj_) in zip(part_idx, bonds):
        msg_ = _tc_node_mlp(bi_, angt16, auxt16, bj_, wg0, w1t, g1t, woutt,
                            bhg0, b1r, g1r, NBLK, e0 // NBLK)
        parts_acc.append(_sc_scatter_add(msg_, d3, zeros_n, n_pad, pc))
    new_node, new_node_pad = _tc_add_partials(node_feat, parts_acc, n_pad,
                                              n_pad // 16)
    bi2, bj2 = _sc_gather2(new_node_pad, idx2, e, nch)
    new_edge_t = _tc_edge_mlp(bi2, angt, auxt16, bj2, wge, we1t, ge1t,
                              bhge, be1r, ge1r, 6400)
    return new_node, new_edge_t.T


# speedup vs baseline: 4.1213x; 1.0025x over previous
import functools

import jax
import jax.numpy as jnp
from jax import lax
from jax.experimental import pallas as pl
from jax.experimental.pallas import tpu as pltpu
from jax.experimental.pallas import tpu_sc as plsc

NC = 2
NS = 16
NW = NC * NS
CH = 128
BF = jnp.bfloat16


def _sc_gather2(table_pad, idx2, e_rows, nch):
    n_pad, d = table_pad.shape
    nw, nt, ch = idx2.shape
    npt = n_pad // NS
    mesh = plsc.VectorSubcoreMesh(core_axis_name="c", subcore_axis_name="s")

    @functools.partial(
        pl.kernel,
        out_type=(
            jax.ShapeDtypeStruct((e_rows, d), jnp.float32),
            jax.ShapeDtypeStruct((e_rows, d), jnp.float32),
        ),
        mesh=mesh,
        scratch_types=[
            pltpu.VMEM((nt, ch), jnp.int32),
            pltpu.VMEM((2, ch, d), jnp.float32),
            pltpu.VMEM_SHARED((n_pad, d), jnp.float32),
            pltpu.SemaphoreType.DMA,
            pltpu.SemaphoreType.DMA,
        ],
    )
    def k(table_h, idx_h, oi_h, oj_h, idx_v, buf_v, tab_s, sem0, sem1):
        ci = lax.axis_index("c")
        si = lax.axis_index("s")
        wid = si * NC + ci
        pltpu.sync_copy(table_h.at[pl.ds(si * npt, npt)],
                        tab_s.at[pl.ds(si * npt, npt)])
        pltpu.sync_copy(idx_h.at[wid], idx_v)
        plsc.subcore_barrier()

        nt_w = 2 * ((nch - wid + NW - 1) // NW)

        def off(k_):
            return (wid + k_ * NW) * ch

        pltpu.async_copy(tab_s.at[idx_v.at[0]], buf_v.at[0], sem0)
        pltpu.async_copy(tab_s.at[idx_v.at[1]], buf_v.at[1], sem1)

        def body(t, carry):
            p = t % 2
            c = t // 2

            @pl.when(p == 0)
            def _even():
                pltpu.make_async_copy(tab_s.at[idx_v.at[t]], buf_v.at[0],
                                      sem0).wait()
                pltpu.sync_copy(buf_v.at[0], oi_h.at[pl.ds(off(c), ch)])
                @pl.when(t + 2 < nt_w)
                def _fire():
                    pltpu.async_copy(tab_s.at[idx_v.at[t + 2]], buf_v.at[0],
                                     sem0)

            @pl.when(p == 1)
            def _odd():
                pltpu.make_async_copy(tab_s.at[idx_v.at[t]], buf_v.at[1],
                                      sem1).wait()
                pltpu.sync_copy(buf_v.at[1], oj_h.at[pl.ds(off(c), ch)])
                @pl.when(t + 2 < nt_w)
                def _fire():
                    pltpu.async_copy(tab_s.at[idx_v.at[t + 2]], buf_v.at[1],
                                     sem1)

            return carry

        lax.fori_loop(0, nt_w, body, 0)

    return k(table_pad, idx2)


def _sc_scatter_add(msg, dst3, zeros, n_pad, nch):
    e_rows, d = msg.shape
    nw, kmax, ch = dst3.shape
    npt = n_pad // NS
    mesh = plsc.VectorSubcoreMesh(core_axis_name="c", subcore_axis_name="s")

    @functools.partial(
        pl.kernel,
        out_type=jax.ShapeDtypeStruct((NC, n_pad, d), jnp.float32),
        mesh=mesh,
        scratch_types=[
            pltpu.VMEM((kmax, ch), jnp.int32),
            pltpu.VMEM((2, ch, d), jnp.float32),
            pltpu.VMEM_SHARED((n_pad, d), jnp.float32),
            pltpu.SemaphoreType.DMA,
        ],
    )
    def k(msg_h, dst_h, zeros_h, out_h, idx_v, rows_v, acc_s, sem):
        ci = lax.axis_index("c")
        si = lax.axis_index("s")
        wid = si * NC + ci
        pltpu.sync_copy(zeros_h.at[pl.ds(si * npt, npt)],
                        acc_s.at[pl.ds(si * npt, npt)])
        plsc.subcore_barrier()
        pltpu.sync_copy(dst_h.at[wid], idx_v)

        nk_w = (nch - wid + NW - 1) // NW

        def off(k_):
            return (wid + k_ * NW) * ch

        pltpu.async_copy(msg_h.at[pl.ds(off(0), ch)], rows_v.at[0], sem)

        def body(c, carry):
            p = c % 2

            @pl.when(c + 1 < nk_w)
            def _fire_next():
                pltpu.async_copy(msg_h.at[pl.ds(off(c + 1), ch)],
                                 rows_v.at[1 - p], sem)

            pltpu.make_async_copy(msg_h.at[pl.ds(off(c), ch)], rows_v.at[p],
                                  sem).wait()
            pltpu.sync_copy(rows_v.at[p], acc_s.at[idx_v.at[c]], add=True)
            return carry

        lax.fori_loop(0, nk_w, body, 0)
        plsc.subcore_barrier()
        pltpu.sync_copy(acc_s.at[pl.ds(si * npt, npt)],
                        out_h.at[ci, pl.ds(si * npt, npt)])

    return k(msg, dst3, zeros)


def _tc_node_mlp(bi, angt, auxt, bj, wg0, w1t, g1t, woutt, bhg, b1r, g1r,
                 block_rows, blk0=0):
    e_rows, dn = bi.shape
    da = angt.shape[0]
    dx = auxt.shape[0]
    h = w1t.shape[0]
    grid = e_rows // block_rows

    def body(bi_r, angt_r, auxt_r, bj_r, wg0_r, w1_r, g1_r, wout_r, bhg_r,
             b1_r, g1b_r, o_r):
        xc = jnp.concatenate(
            [bi_r[...].astype(BF), angt_r[...].T, auxt_r[...].T,
             bj_r[...].astype(BF)], axis=1)
        x = jnp.dot(xc, wg0_r[...], preferred_element_type=jnp.float32)
        x += bhg_r[...]
        a = x * jax.nn.sigmoid(x)
        ah = a[:, :h].astype(BF)
        ag = a[:, h:].astype(BF)
        h2 = jnp.dot(ah, w1_r[...], preferred_element_type=jnp.float32) + b1_r[...]
        h2 = h2 * jax.nn.sigmoid(h2)
        g2 = jax.nn.sigmoid(
            jnp.dot(ag, g1_r[...], preferred_element_type=jnp.float32) + g1b_r[...])
        m = (h2 * g2).astype(BF)
        o_r[...] = jnp.dot(m, wout_r[...], preferred_element_type=jnp.float32)

    full = lambda i: (0, 0)
    return pl.pallas_call(
        body,
        grid=(grid,),
        in_specs=[
            pl.BlockSpec((block_rows, dn), lambda i: (i, 0)),
            pl.BlockSpec((da, block_rows), lambda i: (0, i + blk0)),
            pl.BlockSpec((dx, block_rows), lambda i: (0, i + blk0)),
            pl.BlockSpec((block_rows, dn), lambda i: (i, 0)),
            pl.BlockSpec(wg0.shape, full),
            pl.BlockSpec(w1t.shape, full),
            pl.BlockSpec(g1t.shape, full),
            pl.BlockSpec(woutt.shape, full),
            pl.BlockSpec(bhg.shape, full),
            pl.BlockSpec(b1r.shape, full),
            pl.BlockSpec(g1r.shape, full),
        ],
        out_specs=pl.BlockSpec((block_rows, dn), lambda i: (i, 0)),
        out_shape=jax.ShapeDtypeStruct((e_rows, dn), jnp.float32),
    )(bi, angt, auxt, bj, wg0, w1t, g1t, woutt, bhg, b1r, g1r)


def _tc_edge_mlp(bi, angt, auxt, bj, wge, w1t, g1t, bhg, b1r, g1r,
                 block_rows):
    e_rows, dn = bi.shape
    da = angt.shape[0]
    dx = auxt.shape[0]
    h = w1t.shape[0]
    grid = e_rows // block_rows

    def body(bi_r, angt_r, auxt_r, bj_r, wge_r, w1_r, g1_r, bhg_r, b1_r,
             g1b_r, o_r):
        ang = angt_r[...].T
        xc = jnp.concatenate(
            [bi_r[...].astype(BF), ang.astype(BF), auxt_r[...].T,
             bj_r[...].astype(BF)], axis=1)
        x = jnp.dot(xc, wge_r[...], preferred_element_type=jnp.float32)
        x += bhg_r[...]
        a = x * jax.nn.sigmoid(x)
        ah = a[:, :h].astype(BF)
        ag = a[:, h:].astype(BF)
        h2 = jnp.dot(ah, w1_r[...], preferred_element_type=jnp.float32) + b1_r[...]
        h2 = h2 * jax.nn.sigmoid(h2)
        g2 = jax.nn.sigmoid(
            jnp.dot(ag, g1_r[...], preferred_element_type=jnp.float32) + g1b_r[...])
        o_r[...] = (ang + h2 * g2).T

    full = lambda i: (0, 0)
    return pl.pallas_call(
        body,
        grid=(grid,),
        in_specs=[
            pl.BlockSpec((block_rows, dn), lambda i: (i, 0)),
            pl.BlockSpec((da, block_rows), lambda i: (0, i)),
            pl.BlockSpec((dx, block_rows), lambda i: (0, i)),
            pl.BlockSpec((block_rows, dn), lambda i: (i, 0)),
            pl.BlockSpec(wge.shape, full),
            pl.BlockSpec(w1t.shape, full),
            pl.BlockSpec(g1t.shape, full),
            pl.BlockSpec(bhg.shape, full),
            pl.BlockSpec(b1r.shape, full),
            pl.BlockSpec(g1r.shape, full),
        ],
        out_specs=pl.BlockSpec((da, block_rows), lambda i: (0, i)),
        out_shape=jax.ShapeDtypeStruct((da, e_rows), jnp.float32),
    )(bi, angt, auxt, bj, wge, w1t, g1t, bhg, b1r, g1r)


def _tc_add_partials(node_feat, parts_list, n_pad, block_rows):
    n, d = node_feat.shape
    grid = n_pad // block_rows
    np_ = len(parts_list)

    def body(nf_r, *refs):
        p_refs = refs[:np_]
        o_r, op_r = refs[np_], refs[np_ + 1]
        o = nf_r[...]
        for p_r in p_refs:
            o = o + (p_r[0] + p_r[1])
        o_r[...] = o
        op_r[...] = o

    return pl.pallas_call(
        body,
        grid=(grid,),
        in_specs=[
            pl.BlockSpec((block_rows, d), lambda i: (i, 0)),
        ] + [
            pl.BlockSpec((NC, block_rows, d), lambda i: (0, i, 0))
            for _ in range(np_)
        ],
        out_specs=[
            pl.BlockSpec((block_rows, d), lambda i: (i, 0)),
            pl.BlockSpec((block_rows, d), lambda i: (i, 0)),
        ],
        out_shape=[
            jax.ShapeDtypeStruct((n, d), jnp.float32),
            jax.ShapeDtypeStruct((n_pad, d), jnp.float32),
        ],
    )(node_feat, *parts_list)


def kernel(node_feat, angle_feat, aux_feat, edge_index, W0, b0, W1, b1, G0,
           g0, G1, g1, Wout, We0, be0, We1, be1, Ge0, ge0, Ge1, ge1):
    n, dn = node_feat.shape
    e, da = angle_feat.shape
    dx = aux_feat.shape[1]
    h = W0.shape[0]

    nch = e // CH
    parts_ch = [nch // 5, 2 * nch // 5, 2 * nch // 5]
    kmax = -(-nch // NW)
    n_grain = 8 * NS
    n_pad = n_grain * (-(-n // n_grain))

    src = edge_index[0]
    dst = edge_index[1]

    def chunks3(v, km):
        nc_ = v.shape[0] // CH
        vp = jnp.pad(v, (0, km * NW * CH - v.shape[0]))
        return vp.reshape(km, NW, CH).transpose(1, 0, 2)

    def idx2_of(s_, d_, km):
        s3 = chunks3(s_, km)
        d3 = chunks3(d_, km)
        return jnp.stack([s3, d3], axis=2).reshape(NW, 2 * km, CH), d3

    part_idx = []
    c0 = 0
    for pc in parts_ch:
        e0, e1 = c0 * CH, (c0 + pc) * CH
        km = -(-pc // NW)
        part_idx.append((e0, e1, pc) + idx2_of(src[e0:e1], dst[e0:e1], km))
        c0 += pc
    idx2, _ = idx2_of(src, dst, kmax)

    node_pad = jnp.pad(node_feat, ((0, n_pad - n), (0, 0)))
    zeros_n = jnp.zeros((n_pad, dn), jnp.float32)
    angt = angle_feat.T
    auxt = aux_feat.T
    angt16 = angt.astype(BF)
    auxt16 = auxt.astype(BF)

    wg0 = jnp.concatenate([W0.T, G0.T], axis=1).astype(BF)
    bhg0 = jnp.concatenate([b0, g0]).reshape(1, 2 * h)
    w1t = W1.T.astype(BF)
    g1t = G1.T.astype(BF)
    woutt = Wout.T.astype(BF)
    b1r = b1.reshape(1, -1)
    g1r = g1.reshape(1, -1)

    wge = jnp.concatenate([We0.T, Ge0.T], axis=1).astype(BF)
    bhge = jnp.concatenate([be0, ge0]).reshape(1, 2 * h)
    we1t = We1.T.astype(BF)
    ge1t = Ge1.T.astype(BF)
    be1r = be1.reshape(1, -1)
    ge1r = ge1.reshape(1, -1)

    NBLK = 6400
    bonds = [_sc_gather2(node_pad, p[3], p[1] - p[0], p[2])
             for p in part_idx]
    parts_acc = []
    for (e0, e1, pc, _i2, d3), (bi_, bj_) in zip(part_idx, bonds):
        msg_ = _tc_node_mlp(bi_, angt16, auxt16, bj_, wg0, w1t, g1t, woutt,
                            bhg0, b1r, g1r, NBLK, e0 // NBLK)
        parts_acc.append(_sc_scatter_add(msg_, d3, zeros_n, n_pad, pc))
    new_node, new_node_pad = _tc_add_partials(node_feat, parts_acc, n_pad,
                                              n_pad // 16)
    bi2, bj2 = _sc_gather2(new_node_pad, idx2, e, nch)
    new_edge_t = _tc_edge_mlp(bi2, angt, auxt16, bj2, wge, we1t, ge1t,
                              bhge, be1r, ge1r, 6400)
    return new_node, new_edge_t.T
